# Initial kernel scaffold; baseline (speedup 1.0000x reference)
import jax
import jax.numpy as jnp
from jax.experimental import pallas as pl


def kernel(x, edge_index, node_type_ids, mask_faculty, mask_course, mask_section, mask_room, mask_timeslot, W_faculty, b_faculty, W_course, b_course, W_section, b_section, W_room, b_room, W_timeslot, b_timeslot, sage1_Wl, sage1_Wr, sage1_b, sage2_Wl, sage2_Wr, sage2_b, bn1_gamma, bn1_beta, bn2_gamma, bn2_beta):
    raise NotImplementedError("write your pallas kernel here")


# speedup vs baseline: 5.9344x; 5.9344x over previous
import functools

import jax
import jax.numpy as jnp
from jax import lax
from jax.experimental import pallas as pl
from jax.experimental.pallas import tpu as pltpu
from jax.experimental.pallas import tpu_sc as plsc

N = 100000
E = 1600000
HIDDEN = 64
EMBED = 32
FEATS = [8, 7, 5, 6, 8]

LN = 128
BLK = 8
NC = 2
NT = 16
NW = NC * NT
WROWS = 392
NBLK = WROWS // BLK
RP = NW * WROWS
EP = RP * LN
NP = 100096
NODES_PER_TILE = NP // NT
BN = 2000
GRID = N // BN


def _make_segsum(nchunks, with_deg):
    out_type = [jax.ShapeDtypeStruct((NC * nchunks, NP, 16), jnp.float32)]
    if with_deg:
        out_type.append(jax.ShapeDtypeStruct((NC, NP, 16), jnp.float32))

    scratch = [
        pltpu.VMEM((BLK, LN), jnp.int32),
        pltpu.VMEM((BLK, LN), jnp.int32),
        pltpu.VMEM((BLK, LN, 16), jnp.float32),
        pltpu.VMEM((LN, 16), jnp.float32),
        pltpu.VMEM_SHARED((NP, 16), jnp.float32),
        pltpu.SemaphoreType.DMA,
    ]

    mesh = plsc.VectorSubcoreMesh(core_axis_name="c", subcore_axis_name="s")

    @functools.partial(
        pl.kernel, out_type=out_type, mesh=mesh, scratch_types=scratch,
        compiler_params=pltpu.CompilerParams(use_tc_tiling_on_sc=False))
    def segsum(*refs):
        if with_deg:
            (src2d, dst2d, z2d, *tabs) = refs[:3 + nchunks]
            out, deg_out = refs[3 + nchunks:3 + nchunks + 2]
            scr = refs[3 + nchunks + 2:]
        else:
            (src2d, dst2d, z2d, *tabs) = refs[:3 + nchunks]
            out = refs[3 + nchunks]
            scr = refs[3 + nchunks + 1:]
        (idx_s, idx_d, rows, ones_v, acc, sem) = scr

        c = lax.axis_index("c")
        t = lax.axis_index("s")
        nslc = pl.ds(t * NODES_PER_TILE, NODES_PER_TILE)
        start = (c * NT + t) * WROWS

        if with_deg:
            def fill_ones(i, carry):
                ones_v[i, :] = jnp.ones((16,), jnp.float32)
                return carry
            lax.fori_loop(0, LN, fill_ones, 0)

        passes = ([-1] if with_deg else []) + list(range(nchunks))
        for p in passes:
            pltpu.sync_copy(z2d.at[nslc], acc.at[nslc])
            plsc.subcore_barrier()

            if p < 0:
                def deg_body(i, carry):
                    row0 = start + i * BLK
                    pltpu.sync_copy(dst2d.at[pl.ds(row0, BLK)], idx_d)
                    for j in range(BLK):
                        pltpu.sync_copy(ones_v, acc.at[idx_d.at[j]],
                                        add=True)
                    return carry

                lax.fori_loop(0, NBLK, deg_body, 0)
            else:
                tab = tabs[p]

                def blk_body(i, carry, tab=tab):
                    row0 = start + i * BLK
                    pltpu.sync_copy(src2d.at[pl.ds(row0, BLK)], idx_s)
                    pltpu.sync_copy(dst2d.at[pl.ds(row0, BLK)], idx_d)
                    handles = [pltpu.async_copy(tab.at[idx_s.at[j]],
                                                rows.at[j], sem)
                               for j in range(BLK)]
                    for h in handles:
                        h.wait()
                    for j in range(BLK):
                        pltpu.sync_copy(rows.at[j], acc.at[idx_d.at[j]],
                                        add=True)
                    return carry

                lax.fori_loop(0, NBLK, blk_body, 0)

            plsc.subcore_barrier()
            if p < 0:
                pltpu.sync_copy(acc.at[nslc], deg_out.at[c, nslc])
            else:
                pltpu.sync_copy(acc.at[nslc],
                                out.at[c * nchunks + p, nslc])

    return segsum


def _proj_body(x_ref, tid_ref, w_ref, b_ref, h_ref, c0, c1, c2, c3):
    xb = x_ref[...]
    tid = tid_ref[...]
    acc = jnp.zeros((BN, HIDDEN), jnp.float32)
    for t in range(5):
        z = jnp.dot(xb, w_ref[t], preferred_element_type=jnp.float32)
        z = jnp.maximum(z + b_ref[t], 0.0)
        acc = jnp.where(tid == t, z, acc)
    h_ref[...] = acc
    c0[...] = acc[:, 0:16]
    c1[...] = acc[:, 16:32]
    c2[...] = acc[:, 32:48]
    c3[...] = acc[:, 48:64]


def _combine1_body(h_ref, agg_ref, deg_ref, wl_ref, wr_ref, b_ref,
                   y_ref, s1_ref, s2_ref):
    a = agg_ref[...]
    asum = a[0:4] + a[4:8]
    m = jnp.concatenate([asum[0], asum[1], asum[2], asum[3]], axis=1)
    d = jnp.maximum(deg_ref[0, :, 0] + deg_ref[1, :, 0], 1.0)
    mean = m / d[:, None]
    y = (jnp.dot(mean, wl_ref[...], preferred_element_type=jnp.float32)
         + jnp.dot(h_ref[...], wr_ref[...],
                   preferred_element_type=jnp.float32)
         + b_ref[...])
    y_ref[...] = y

    @pl.when(pl.program_id(0) == 0)
    def _():
        s1_ref[...] = jnp.zeros_like(s1_ref)
        s2_ref[...] = jnp.zeros_like(s2_ref)

    s1_ref[...] += jnp.sum(y, axis=0, keepdims=True)
    s2_ref[...] += jnp.sum(y * y, axis=0, keepdims=True)


def _bn1_body(y_ref, s1_ref, s2_ref, g_ref, be_ref, wl2_ref,
              h1_ref, g0_ref, g1_ref):
    mu = s1_ref[...] / N
    var = s2_ref[...] / N - mu * mu
    inv = lax.rsqrt(var + 1e-5)
    h1 = g_ref[...] * (y_ref[...] - mu) * inv + be_ref[...]
    h1 = jnp.maximum(h1, 0.0)
    h1_ref[...] = h1
    g = jnp.dot(h1, wl2_ref[...], preferred_element_type=jnp.float32)
    g0_ref[...] = g[:, 0:16]
    g1_ref[...] = g[:, 16:32]


def _combine2_body(h1_ref, agg_ref, deg_ref, wr2_ref, b2_ref,
                   y_ref, s1_ref, s2_ref):
    a = agg_ref[...]
    asum = a[0:2] + a[2:4]
    m = jnp.concatenate([asum[0], asum[1]], axis=1)
    d = jnp.maximum(deg_ref[0, :, 0] + deg_ref[1, :, 0], 1.0)
    y = (m / d[:, None]
         + jnp.dot(h1_ref[...], wr2_ref[...],
                   preferred_element_type=jnp.float32)
         + b2_ref[...])
    y_ref[...] = y

    @pl.when(pl.program_id(0) == 0)
    def _():
        s1_ref[...] = jnp.zeros_like(s1_ref)
        s2_ref[...] = jnp.zeros_like(s2_ref)

    s1_ref[...] += jnp.sum(y, axis=0, keepdims=True)
    s2_ref[...] += jnp.sum(y * y, axis=0, keepdims=True)


def _bn2_body(y_ref, s1_ref, s2_ref, g_ref, be_ref, out_ref):
    mu = s1_ref[...] / N
    var = s2_ref[...] / N - mu * mu
    inv = lax.rsqrt(var + 1e-5)
    out_ref[...] = g_ref[...] * (y_ref[...] - mu) * inv + be_ref[...]


def _full(shape):
    return pl.BlockSpec(shape, lambda i: tuple(0 for _ in shape))


def _rows(shape):
    def imap(i):
        return (i,) + tuple(0 for _ in shape[1:])
    return pl.BlockSpec(shape, imap)


def _mid(shape):
    def imap(i):
        return (0, i) + tuple(0 for _ in shape[2:])
    return pl.BlockSpec(shape, imap)


def kernel(x, edge_index, node_type_ids, mask_faculty, mask_course,
           mask_section, mask_room, mask_timeslot, W_faculty, b_faculty,
           W_course, b_course, W_section, b_section, W_room, b_room,
           W_timeslot, b_timeslot, sage1_Wl, sage1_Wr, sage1_b,
           sage2_Wl, sage2_Wr, sage2_b, bn1_gamma, bn1_beta,
           bn2_gamma, bn2_beta):
    f32 = jnp.float32
    Ws = [W_faculty, W_course, W_section, W_room, W_timeslot]
    bs = [b_faculty, b_course, b_section, b_room, b_timeslot]
    wpad = jnp.stack([jnp.pad(w, ((0, 8 - w.shape[0]), (0, 0)))
                      for w in Ws])
    bstack = jnp.stack(bs)
    tid2d = node_type_ids.reshape(N, 1)

    npad = EP - E
    pad_src = (jnp.arange(npad, dtype=jnp.int32) * 17) % N
    pad_dst = N + (jnp.arange(npad, dtype=jnp.int32) % (NP - N))
    src2d = jnp.concatenate([edge_index[0], pad_src]).reshape(RP, LN)
    dst2d = jnp.concatenate([edge_index[1], pad_dst]).reshape(RP, LN)
    z2d = jnp.zeros((NP, 16), f32)

    h, hc0, hc1, hc2, hc3 = pl.pallas_call(
        _proj_body,
        grid=(GRID,),
        in_specs=[_rows((BN, 8)), _rows((BN, 1)),
                  _full((5, 8, HIDDEN)), _full((5, HIDDEN))],
        out_specs=[_rows((BN, HIDDEN))] + [_rows((BN, 16))] * 4,
        out_shape=[jax.ShapeDtypeStruct((N, HIDDEN), f32)]
        + [jax.ShapeDtypeStruct((N, 16), f32)] * 4,
    )(x, tid2d, wpad, bstack)

    agg1, degp = _make_segsum(4, True)(src2d, dst2d, z2d,
                                       hc0, hc1, hc2, hc3)

    y1, s1, s2 = pl.pallas_call(
        _combine1_body,
        grid=(GRID,),
        in_specs=[_rows((BN, HIDDEN)), _mid((8, BN, 16)),
                  _mid((NC, BN, 16)), _full((HIDDEN, HIDDEN)),
                  _full((HIDDEN, HIDDEN)), _full((1, HIDDEN))],
        out_specs=[_rows((BN, HIDDEN)), _full((1, HIDDEN)),
                   _full((1, HIDDEN))],
        out_shape=[jax.ShapeDtypeStruct((N, HIDDEN), f32),
                   jax.ShapeDtypeStruct((1, HIDDEN), f32),
                   jax.ShapeDtypeStruct((1, HIDDEN), f32)],
    )(h, agg1, degp, sage1_Wl, sage1_Wr, sage1_b.reshape(1, HIDDEN))

    h1, g0, g1 = pl.pallas_call(
        _bn1_body,
        grid=(GRID,),
        in_specs=[_rows((BN, HIDDEN)), _full((1, HIDDEN)),
                  _full((1, HIDDEN)), _full((1, HIDDEN)),
                  _full((1, HIDDEN)), _full((HIDDEN, EMBED))],
        out_specs=[_rows((BN, HIDDEN)), _rows((BN, 16)),
                   _rows((BN, 16))],
        out_shape=[jax.ShapeDtypeStruct((N, HIDDEN), f32),
                   jax.ShapeDtypeStruct((N, 16), f32),
                   jax.ShapeDtypeStruct((N, 16), f32)],
    )(y1, s1, s2, bn1_gamma.reshape(1, HIDDEN),
      bn1_beta.reshape(1, HIDDEN), sage2_Wl)

    (agg2,) = _make_segsum(2, False)(src2d, dst2d, z2d, g0, g1)

    y2, t1, t2 = pl.pallas_call(
        _combine2_body,
        grid=(GRID,),
        in_specs=[_rows((BN, HIDDEN)), _mid((4, BN, 16)),
                  _mid((NC, BN, 16)), _full((HIDDEN, EMBED)),
                  _full((1, EMBED))],
        out_specs=[_rows((BN, EMBED)), _full((1, EMBED)),
                   _full((1, EMBED))],
        out_shape=[jax.ShapeDtypeStruct((N, EMBED), f32),
                   jax.ShapeDtypeStruct((1, EMBED), f32),
                   jax.ShapeDtypeStruct((1, EMBED), f32)],
    )(h1, agg2, degp, sage2_Wr, sage2_b.reshape(1, EMBED))

    out = pl.pallas_call(
        _bn2_body,
        grid=(GRID,),
        in_specs=[_rows((BN, EMBED)), _full((1, EMBED)),
                  _full((1, EMBED)), _full((1, EMBED)),
                  _full((1, EMBED))],
        out_specs=_rows((BN, EMBED)),
        out_shape=jax.ShapeDtypeStruct((N, EMBED), f32),
    )(y2, t1, t2, bn2_gamma.reshape(1, EMBED),
      bn2_beta.reshape(1, EMBED))

    return out


# speedup vs baseline: 7.6475x; 1.2887x over previous
import functools

import jax
import jax.numpy as jnp
from jax import lax
from jax.experimental import pallas as pl
from jax.experimental.pallas import tpu as pltpu
from jax.experimental.pallas import tpu_sc as plsc

N = 100000
E = 1600000
HIDDEN = 64
EMBED = 32
FEATS = [8, 7, 5, 6, 8]

LN = 128
BLK = 5
NC = 2
NT = 16
NW = NC * NT
WROWS = 400
NBLK = WROWS // BLK
PAIRS = NBLK // 2
RP = NW * WROWS
RP_ALLOC = RP + 4 * BLK
EP = RP_ALLOC * LN
NP = 100096
NODES_PER_TILE = NP // NT
BN = 2000
GRID = N // BN


def _make_segsum(nchunks, with_deg):
    out_type = [jax.ShapeDtypeStruct((NC * nchunks, NP, 16), jnp.float32)]
    if with_deg:
        out_type.append(jax.ShapeDtypeStruct((NC, NP, 16), jnp.float32))

    scratch = [
        pltpu.VMEM((BLK, LN), jnp.int32),
        pltpu.VMEM((BLK, LN), jnp.int32),
        pltpu.VMEM((BLK, LN), jnp.int32),
        pltpu.VMEM((BLK, LN), jnp.int32),
        pltpu.VMEM((BLK, LN, 16), jnp.float32),
        pltpu.VMEM((BLK, LN, 16), jnp.float32),
        pltpu.VMEM((LN, 16), jnp.float32),
        pltpu.VMEM_SHARED((NP, 16), jnp.float32),
        pltpu.SemaphoreType.DMA,
        pltpu.SemaphoreType.DMA,
        pltpu.SemaphoreType.DMA,
        pltpu.SemaphoreType.DMA,
        pltpu.SemaphoreType.DMA,
    ]

    mesh = plsc.VectorSubcoreMesh(core_axis_name="c", subcore_axis_name="s")

    @functools.partial(
        pl.kernel, out_type=out_type, mesh=mesh, scratch_types=scratch,
        compiler_params=pltpu.CompilerParams(use_tc_tiling_on_sc=False))
    def segsum(*refs):
        if with_deg:
            (src2d, dst2d, z2d, *tabs) = refs[:3 + nchunks]
            out, deg_out = refs[3 + nchunks:3 + nchunks + 2]
            scr = refs[3 + nchunks + 2:]
        else:
            (src2d, dst2d, z2d, *tabs) = refs[:3 + nchunks]
            out = refs[3 + nchunks]
            scr = refs[3 + nchunks + 1:]
        (idxA_s, idxA_d, idxB_s, idxB_d, rowsA, rowsB, ones_v, acc,
         semGA, semGB, semIA, semIB, semS) = scr

        c = lax.axis_index("c")
        t = lax.axis_index("s")
        nslc = pl.ds(t * NODES_PER_TILE, NODES_PER_TILE)
        start = (c * NT + t) * WROWS

        def idx_start(row0, i_s, i_d, sem):
            pltpu.async_copy(src2d.at[pl.ds(row0, BLK)], i_s, sem)
            pltpu.async_copy(dst2d.at[pl.ds(row0, BLK)], i_d, sem)

        def idx_wait(i_s, i_d, sem):
            pltpu.make_async_copy(src2d.at[pl.ds(0, BLK)], i_s,
                                  sem).wait()
            pltpu.make_async_copy(dst2d.at[pl.ds(0, BLK)], i_d,
                                  sem).wait()

        def fire_g(tab, i_s, rows, sem):
            for j in range(BLK):
                pltpu.async_copy(tab.at[i_s.at[j]], rows.at[j], sem)

        def drain_g(tab, i_s, rows, sem):
            for j in range(BLK):
                pltpu.make_async_copy(tab.at[i_s.at[j]], rows.at[j],
                                      sem).wait()

        def scatter(vals, i_d):
            hs = [pltpu.async_copy(vals.at[j], acc.at[i_d.at[j]],
                                   semS, add=True) for j in range(BLK)]
            for h in hs:
                h.wait()

        def scatter_ones(i_d):
            hs = [pltpu.async_copy(ones_v, acc.at[i_d.at[j]],
                                   semS, add=True) for j in range(BLK)]
            for h in hs:
                h.wait()

        if with_deg:
            def fill_ones(i, carry):
                ones_v[i, :] = jnp.ones((16,), jnp.float32)
                return carry
            lax.fori_loop(0, LN, fill_ones, 0)

        passes = ([-1] if with_deg else []) + list(range(nchunks))
        for p in passes:
            pltpu.sync_copy(z2d.at[nslc], acc.at[nslc])
            plsc.subcore_barrier()

            if p < 0:
                pltpu.sync_copy(dst2d.at[pl.ds(start, BLK)], idxA_d)
                pltpu.async_copy(dst2d.at[pl.ds(start + BLK, BLK)],
                                 idxB_d, semIB)

                def deg_body(k, carry):
                    b0 = start + (2 * k) * BLK
                    scatter_ones(idxA_d)
                    pltpu.async_copy(
                        dst2d.at[pl.ds(b0 + 2 * BLK, BLK)], idxA_d,
                        semIA)
                    pltpu.make_async_copy(
                        dst2d.at[pl.ds(0, BLK)], idxB_d, semIB).wait()
                    scatter_ones(idxB_d)
                    pltpu.async_copy(
                        dst2d.at[pl.ds(b0 + 3 * BLK, BLK)], idxB_d,
                        semIB)
                    pltpu.make_async_copy(
                        dst2d.at[pl.ds(0, BLK)], idxA_d, semIA).wait()
                    return carry

                lax.fori_loop(0, PAIRS, deg_body, 0)
                pltpu.make_async_copy(dst2d.at[pl.ds(0, BLK)], idxB_d,
                                      semIB).wait()
            else:
                tab = tabs[p]
                pltpu.sync_copy(src2d.at[pl.ds(start, BLK)], idxA_s)
                pltpu.sync_copy(dst2d.at[pl.ds(start, BLK)], idxA_d)
                fire_g(tab, idxA_s, rowsA, semGA)
                idx_start(start + BLK, idxB_s, idxB_d, semIB)

                def pair_body(k, carry, tab=tab):
                    b0 = start + (2 * k) * BLK
                    idx_wait(idxB_s, idxB_d, semIB)
                    fire_g(tab, idxB_s, rowsB, semGB)
                    drain_g(tab, idxA_s, rowsA, semGA)
                    scatter(rowsA, idxA_d)
                    idx_start(b0 + 2 * BLK, idxA_s, idxA_d, semIA)
                    idx_wait(idxA_s, idxA_d, semIA)
                    fire_g(tab, idxA_s, rowsA, semGA)
                    drain_g(tab, idxB_s, rowsB, semGB)
                    scatter(rowsB, idxB_d)
                    idx_start(b0 + 3 * BLK, idxB_s, idxB_d, semIB)
                    return carry

                lax.fori_loop(0, PAIRS, pair_body, 0)
                idx_wait(idxB_s, idxB_d, semIB)
                drain_g(tab, idxA_s, rowsA, semGA)

            plsc.subcore_barrier()
            if p < 0:
                pltpu.sync_copy(acc.at[nslc], deg_out.at[c, nslc])
            else:
                pltpu.sync_copy(acc.at[nslc],
                                out.at[c * nchunks + p, nslc])

    return segsum


def _proj_body(x_ref, tid_ref, w_ref, b_ref, h_ref, c0, c1, c2, c3):
    xb = x_ref[...]
    tid = tid_ref[...]
    acc = jnp.zeros((BN, HIDDEN), jnp.float32)
    for t in range(5):
        z = jnp.dot(xb, w_ref[t], preferred_element_type=jnp.float32)
        z = jnp.maximum(z + b_ref[t], 0.0)
        acc = jnp.where(tid == t, z, acc)
    h_ref[...] = acc
    c0[...] = acc[:, 0:16]
    c1[...] = acc[:, 16:32]
    c2[...] = acc[:, 32:48]
    c3[...] = acc[:, 48:64]


def _combine1_body(h_ref, agg_ref, deg_ref, wl_ref, wr_ref, b_ref,
                   y_ref, s1_ref, s2_ref):
    a = agg_ref[...]
    asum = a[0:4] + a[4:8]
    m = jnp.concatenate([asum[0], asum[1], asum[2], asum[3]], axis=1)
    d = jnp.maximum(deg_ref[0, :, 0] + deg_ref[1, :, 0], 1.0)
    mean = m / d[:, None]
    y = (jnp.dot(mean, wl_ref[...], preferred_element_type=jnp.float32)
         + jnp.dot(h_ref[...], wr_ref[...],
                   preferred_element_type=jnp.float32)
         + b_ref[...])
    y_ref[...] = y

    @pl.when(pl.program_id(0) == 0)
    def _():
        s1_ref[...] = jnp.zeros_like(s1_ref)
        s2_ref[...] = jnp.zeros_like(s2_ref)

    s1_ref[...] += jnp.sum(y, axis=0, keepdims=True)
    s2_ref[...] += jnp.sum(y * y, axis=0, keepdims=True)


def _bn1_body(y_ref, s1_ref, s2_ref, g_ref, be_ref, wl2_ref,
              h1_ref, g0_ref, g1_ref):
    mu = s1_ref[...] / N
    var = s2_ref[...] / N - mu * mu
    inv = lax.rsqrt(var + 1e-5)
    h1 = g_ref[...] * (y_ref[...] - mu) * inv + be_ref[...]
    h1 = jnp.maximum(h1, 0.0)
    h1_ref[...] = h1
    g = jnp.dot(h1, wl2_ref[...], preferred_element_type=jnp.float32)
    g0_ref[...] = g[:, 0:16]
    g1_ref[...] = g[:, 16:32]


def _combine2_body(h1_ref, agg_ref, deg_ref, wr2_ref, b2_ref,
                   y_ref, s1_ref, s2_ref):
    a = agg_ref[...]
    asum = a[0:2] + a[2:4]
    m = jnp.concatenate([asum[0], asum[1]], axis=1)
    d = jnp.maximum(deg_ref[0, :, 0] + deg_ref[1, :, 0], 1.0)
    y = (m / d[:, None]
         + jnp.dot(h1_ref[...], wr2_ref[...],
                   preferred_element_type=jnp.float32)
         + b2_ref[...])
    y_ref[...] = y

    @pl.when(pl.program_id(0) == 0)
    def _():
        s1_ref[...] = jnp.zeros_like(s1_ref)
        s2_ref[...] = jnp.zeros_like(s2_ref)

    s1_ref[...] += jnp.sum(y, axis=0, keepdims=True)
    s2_ref[...] += jnp.sum(y * y, axis=0, keepdims=True)


def _bn2_body(y_ref, s1_ref, s2_ref, g_ref, be_ref, out_ref):
    mu = s1_ref[...] / N
    var = s2_ref[...] / N - mu * mu
    inv = lax.rsqrt(var + 1e-5)
    out_ref[...] = g_ref[...] * (y_ref[...] - mu) * inv + be_ref[...]


def _full(shape):
    return pl.BlockSpec(shape, lambda i: tuple(0 for _ in shape))


def _rows(shape):
    def imap(i):
        return (i,) + tuple(0 for _ in shape[1:])
    return pl.BlockSpec(shape, imap)


def _mid(shape):
    def imap(i):
        return (0, i) + tuple(0 for _ in shape[2:])
    return pl.BlockSpec(shape, imap)


def kernel(x, edge_index, node_type_ids, mask_faculty, mask_course,
           mask_section, mask_room, mask_timeslot, W_faculty, b_faculty,
           W_course, b_course, W_section, b_section, W_room, b_room,
           W_timeslot, b_timeslot, sage1_Wl, sage1_Wr, sage1_b,
           sage2_Wl, sage2_Wr, sage2_b, bn1_gamma, bn1_beta,
           bn2_gamma, bn2_beta):
    f32 = jnp.float32
    Ws = [W_faculty, W_course, W_section, W_room, W_timeslot]
    bs = [b_faculty, b_course, b_section, b_room, b_timeslot]
    wpad = jnp.stack([jnp.pad(w, ((0, 8 - w.shape[0]), (0, 0)))
                      for w in Ws])
    bstack = jnp.stack(bs)
    tid2d = node_type_ids.reshape(N, 1)

    npad = EP - E
    pad_src = (jnp.arange(npad, dtype=jnp.int32) * 17) % N
    pad_dst = N + (jnp.arange(npad, dtype=jnp.int32) % (NP - N))
    src2d = jnp.concatenate([edge_index[0], pad_src]).reshape(RP_ALLOC, LN)
    dst2d = jnp.concatenate([edge_index[1], pad_dst]).reshape(RP_ALLOC, LN)
    z2d = jnp.zeros((NP, 16), f32)

    h, hc0, hc1, hc2, hc3 = pl.pallas_call(
        _proj_body,
        grid=(GRID,),
        in_specs=[_rows((BN, 8)), _rows((BN, 1)),
                  _full((5, 8, HIDDEN)), _full((5, HIDDEN))],
        out_specs=[_rows((BN, HIDDEN))] + [_rows((BN, 16))] * 4,
        out_shape=[jax.ShapeDtypeStruct((N, HIDDEN), f32)]
        + [jax.ShapeDtypeStruct((N, 16), f32)] * 4,
    )(x, tid2d, wpad, bstack)

    agg1, degp = _make_segsum(4, True)(src2d, dst2d, z2d,
                                       hc0, hc1, hc2, hc3)

    y1, s1, s2 = pl.pallas_call(
        _combine1_body,
        grid=(GRID,),
        in_specs=[_rows((BN, HIDDEN)), _mid((8, BN, 16)),
                  _mid((NC, BN, 16)), _full((HIDDEN, HIDDEN)),
                  _full((HIDDEN, HIDDEN)), _full((1, HIDDEN))],
        out_specs=[_rows((BN, HIDDEN)), _full((1, HIDDEN)),
                   _full((1, HIDDEN))],
        out_shape=[jax.ShapeDtypeStruct((N, HIDDEN), f32),
                   jax.ShapeDtypeStruct((1, HIDDEN), f32),
                   jax.ShapeDtypeStruct((1, HIDDEN), f32)],
    )(h, agg1, degp, sage1_Wl, sage1_Wr, sage1_b.reshape(1, HIDDEN))

    h1, g0, g1 = pl.pallas_call(
        _bn1_body,
        grid=(GRID,),
        in_specs=[_rows((BN, HIDDEN)), _full((1, HIDDEN)),
                  _full((1, HIDDEN)), _full((1, HIDDEN)),
                  _full((1, HIDDEN)), _full((HIDDEN, EMBED))],
        out_specs=[_rows((BN, HIDDEN)), _rows((BN, 16)),
                   _rows((BN, 16))],
        out_shape=[jax.ShapeDtypeStruct((N, HIDDEN), f32),
                   jax.ShapeDtypeStruct((N, 16), f32),
                   jax.ShapeDtypeStruct((N, 16), f32)],
    )(y1, s1, s2, bn1_gamma.reshape(1, HIDDEN),
      bn1_beta.reshape(1, HIDDEN), sage2_Wl)

    (agg2,) = _make_segsum(2, False)(src2d, dst2d, z2d, g0, g1)

    y2, t1, t2 = pl.pallas_call(
        _combine2_body,
        grid=(GRID,),
        in_specs=[_rows((BN, HIDDEN)), _mid((4, BN, 16)),
                  _mid((NC, BN, 16)), _full((HIDDEN, EMBED)),
                  _full((1, EMBED))],
        out_specs=[_rows((BN, EMBED)), _full((1, EMBED)),
                   _full((1, EMBED))],
        out_shape=[jax.ShapeDtypeStruct((N, EMBED), f32),
                   jax.ShapeDtypeStruct((1, EMBED), f32),
                   jax.ShapeDtypeStruct((1, EMBED), f32)],
    )(h1, agg2, degp, sage2_Wr, sage2_b.reshape(1, EMBED))

    out = pl.pallas_call(
        _bn2_body,
        grid=(GRID,),
        in_specs=[_rows((BN, EMBED)), _full((1, EMBED)),
                  _full((1, EMBED)), _full((1, EMBED)),
                  _full((1, EMBED))],
        out_specs=_rows((BN, EMBED)),
        out_shape=jax.ShapeDtypeStruct((N, EMBED), f32),
    )(y2, t1, t2, bn2_gamma.reshape(1, EMBED),
      bn2_beta.reshape(1, EMBED))

    return out


# speedup vs baseline: 11.9378x; 1.5610x over previous
import functools

import jax
import jax.numpy as jnp
from jax import lax
from jax.experimental import pallas as pl
from jax.experimental.pallas import tpu as pltpu
from jax.experimental.pallas import tpu_sc as plsc

N = 100000
E = 1600000
HIDDEN = 64
EMBED = 32

LN = 128
BLK = 5
NC = 2
NT = 16
NW = NC * NT
WROWS = 400
NBLK = WROWS // BLK
PAIRS = NBLK // 2
RP = NW * WROWS
RP_ALLOC = RP + 4 * BLK
EP = RP_ALLOC * LN

NP = 102400
NODES_PER_TILE = NP // NT
BR = 256
GRID = NP // (8 * BR)
NREAL = N // 8


def _make_segsum(nchunks, with_deg):
    out_type = [jax.ShapeDtypeStruct((NC * nchunks, NP, 16), jnp.float32)]
    if with_deg:
        out_type.append(jax.ShapeDtypeStruct((NC, NP, 16), jnp.float32))

    scratch = [
        pltpu.VMEM((BLK, LN), jnp.int32),
        pltpu.VMEM((BLK, LN), jnp.int32),
        pltpu.VMEM((BLK, LN), jnp.int32),
        pltpu.VMEM((BLK, LN), jnp.int32),
        pltpu.VMEM((BLK, LN, 16), jnp.float32),
        pltpu.VMEM((BLK, LN, 16), jnp.float32),
        pltpu.VMEM((LN, 16), jnp.float32),
        pltpu.VMEM_SHARED((NP, 16), jnp.float32),
        pltpu.SemaphoreType.DMA,
        pltpu.SemaphoreType.DMA,
        pltpu.SemaphoreType.DMA,
        pltpu.SemaphoreType.DMA,
        pltpu.SemaphoreType.DMA,
    ]

    mesh = plsc.VectorSubcoreMesh(core_axis_name="c", subcore_axis_name="s")

    @functools.partial(
        pl.kernel, out_type=out_type, mesh=mesh, scratch_types=scratch,
        compiler_params=pltpu.CompilerParams(use_tc_tiling_on_sc=False))
    def segsum(*refs):
        (src2d, dst2d, z2d, *tabs) = refs[:3 + nchunks]
        if with_deg:
            out, deg_out = refs[3 + nchunks:3 + nchunks + 2]
            scr = refs[3 + nchunks + 2:]
        else:
            out = refs[3 + nchunks]
            scr = refs[3 + nchunks + 1:]
        (idxA_s, idxA_d, idxB_s, idxB_d, rowsA, rowsB, ones_v, acc,
         semGA, semGB, semIA, semIB, semS) = scr

        c = lax.axis_index("c")
        t = lax.axis_index("s")
        nslc = pl.ds(t * NODES_PER_TILE, NODES_PER_TILE)
        start = (c * NT + t) * WROWS

        def idx_start(row0, i_s, i_d, sem):
            pltpu.async_copy(src2d.at[pl.ds(row0, BLK)], i_s, sem)
            pltpu.async_copy(dst2d.at[pl.ds(row0, BLK)], i_d, sem)

        def idx_wait(i_s, i_d, sem):
            pltpu.make_async_copy(src2d.at[pl.ds(0, BLK)], i_s,
                                  sem).wait()
            pltpu.make_async_copy(dst2d.at[pl.ds(0, BLK)], i_d,
                                  sem).wait()

        def fire_g(tab, i_s, rows, sem):
            for j in range(BLK):
                pltpu.async_copy(tab.at[i_s.at[j]], rows.at[j], sem)

        def drain_g(tab, i_s, rows, sem):
            for j in range(BLK):
                pltpu.make_async_copy(tab.at[i_s.at[j]], rows.at[j],
                                      sem).wait()

        def scatter(vals, i_d):
            hs = [pltpu.async_copy(vals.at[j], acc.at[i_d.at[j]],
                                   semS, add=True) for j in range(BLK)]
            for h in hs:
                h.wait()

        def scatter_ones(i_d):
            hs = [pltpu.async_copy(ones_v, acc.at[i_d.at[j]],
                                   semS, add=True) for j in range(BLK)]
            for h in hs:
                h.wait()

        if with_deg:
            def fill_ones(i, carry):
                ones_v[i, :] = jnp.ones((16,), jnp.float32)
                return carry
            lax.fori_loop(0, LN, fill_ones, 0)

        passes = ([-1] if with_deg else []) + list(range(nchunks))
        for p in passes:
            pltpu.sync_copy(z2d.at[nslc], acc.at[nslc])
            plsc.subcore_barrier()

            if p < 0:
                pltpu.sync_copy(dst2d.at[pl.ds(start, BLK)], idxA_d)
                pltpu.async_copy(dst2d.at[pl.ds(start + BLK, BLK)],
                                 idxB_d, semIB)

                def deg_body(k, carry):
                    b0 = start + (2 * k) * BLK
                    scatter_ones(idxA_d)
                    pltpu.async_copy(
                        dst2d.at[pl.ds(b0 + 2 * BLK, BLK)], idxA_d,
                        semIA)
                    pltpu.make_async_copy(
                        dst2d.at[pl.ds(0, BLK)], idxB_d, semIB).wait()
                    scatter_ones(idxB_d)
                    pltpu.async_copy(
                        dst2d.at[pl.ds(b0 + 3 * BLK, BLK)], idxB_d,
                        semIB)
                    pltpu.make_async_copy(
                        dst2d.at[pl.ds(0, BLK)], idxA_d, semIA).wait()
                    return carry

                lax.fori_loop(0, PAIRS, deg_body, 0)
                pltpu.make_async_copy(dst2d.at[pl.ds(0, BLK)], idxB_d,
                                      semIB).wait()
            else:
                tab = tabs[p]
                pltpu.sync_copy(src2d.at[pl.ds(start, BLK)], idxA_s)
                pltpu.sync_copy(dst2d.at[pl.ds(start, BLK)], idxA_d)
                fire_g(tab, idxA_s, rowsA, semGA)
                idx_start(start + BLK, idxB_s, idxB_d, semIB)

                def pair_body(k, carry, tab=tab):
                    b0 = start + (2 * k) * BLK
                    idx_wait(idxB_s, idxB_d, semIB)
                    fire_g(tab, idxB_s, rowsB, semGB)
                    drain_g(tab, idxA_s, rowsA, semGA)
                    scatter(rowsA, idxA_d)
                    idx_start(b0 + 2 * BLK, idxA_s, idxA_d, semIA)
                    idx_wait(idxA_s, idxA_d, semIA)
                    fire_g(tab, idxA_s, rowsA, semGA)
                    drain_g(tab, idxB_s, rowsB, semGB)
                    scatter(rowsB, idxB_d)
                    idx_start(b0 + 3 * BLK, idxB_s, idxB_d, semIB)
                    return carry

                lax.fori_loop(0, PAIRS, pair_body, 0)
                idx_wait(idxB_s, idxB_d, semIB)
                drain_g(tab, idxA_s, rowsA, semGA)

            plsc.subcore_barrier()
            if p < 0:
                pltpu.sync_copy(acc.at[nslc], deg_out.at[c, nslc])
            else:
                pltpu.sync_copy(acc.at[nslc],
                                out.at[c * nchunks + p, nslc])

    return segsum


def _proj_body(x_ref, tid_ref, w_ref, b_ref, ep_ref,
               h_ref, c0, c1, c2, c3):
    xb = x_ref[0]
    tid = tid_ref[0]
    acc = jnp.zeros((BR, 8 * HIDDEN), jnp.float32)
    for t in range(5):
        z = jnp.dot(xb, w_ref[t], preferred_element_type=jnp.float32)
        z = jnp.maximum(z + b_ref[t], 0.0)
        acc = jnp.where(tid == t, z, acc)
    h_ref[...] = acc.reshape(1, BR, 8 * HIDDEN)
    outs = [c0, c1, c2, c3]
    for p in range(4):
        cp = jnp.dot(acc, ep_ref[p], preferred_element_type=jnp.float32)
        outs[p][...] = cp.reshape(1, BR, 128)


def _combine1_body(h_ref, agg_ref, deg_ref, wl_ref, wr_ref, b_ref,
                   y_ref, s1_ref, s2_ref):
    a = agg_ref[...]
    d = jnp.maximum(deg_ref[0, 0] + deg_ref[1, 0], 1.0)
    mp = [(a[p, 0] + a[4 + p, 0]) / d for p in range(4)]
    mcat = jnp.concatenate(mp, axis=1)
    y = (jnp.dot(mcat, wl_ref[...], preferred_element_type=jnp.float32)
         + jnp.dot(h_ref[0], wr_ref[...],
                   preferred_element_type=jnp.float32)
         + b_ref[...])
    y_ref[...] = y.reshape(1, BR, 8 * HIDDEN)

    @pl.when(pl.program_id(0) == 0)
    def _():
        s1_ref[...] = jnp.zeros_like(s1_ref)
        s2_ref[...] = jnp.zeros_like(s2_ref)

    rid = lax.broadcasted_iota(jnp.int32, (BR, 1), 0)
    real = (pl.program_id(0) * BR + rid) < NREAL
    ym = jnp.where(real, y, 0.0)
    s1_ref[...] += jnp.sum(ym, axis=0, keepdims=True)
    s2_ref[...] += jnp.sum(ym * ym, axis=0, keepdims=True)


def _fold8(s, width):
    parts = [s[:, k * width:(k + 1) * width] for k in range(8)]
    tot = parts[0]
    for q in parts[1:]:
        tot = tot + q
    return tot


def _bn1_body(y_ref, s1_ref, s2_ref, g_ref, be_ref, w2a_ref, w2b_ref,
              h1_ref, g0_ref, g1_ref):
    mu = _fold8(s1_ref[...], HIDDEN) / N
    var = _fold8(s2_ref[...], HIDDEN) / N - mu * mu
    inv = lax.rsqrt(var + 1e-5)
    mu8 = jnp.concatenate([mu] * 8, axis=1)
    inv8 = jnp.concatenate([inv] * 8, axis=1)
    h1 = g_ref[...] * (y_ref[0] - mu8) * inv8 + be_ref[...]
    h1 = jnp.maximum(h1, 0.0)
    h1_ref[...] = h1.reshape(1, BR, 8 * HIDDEN)
    g0 = jnp.dot(h1, w2a_ref[...], preferred_element_type=jnp.float32)
    g1 = jnp.dot(h1, w2b_ref[...], preferred_element_type=jnp.float32)
    g0_ref[...] = g0.reshape(1, BR, 128)
    g1_ref[...] = g1.reshape(1, BR, 128)


def _combine2_body(h1_ref, agg_ref, deg_ref, wr2_ref, b2_ref, p_ref,
                   y_ref, s1_ref, s2_ref):
    a = agg_ref[...]
    d = jnp.maximum(deg_ref[0, 0] + deg_ref[1, 0], 1.0)
    mp = [(a[p, 0] + a[2 + p, 0]) / d for p in range(2)]
    mcat = jnp.concatenate(mp, axis=1)
    y = (jnp.dot(mcat, p_ref[...], preferred_element_type=jnp.float32)
         + jnp.dot(h1_ref[0], wr2_ref[...],
                   preferred_element_type=jnp.float32)
         + b2_ref[...])
    y_ref[...] = y.reshape(1, BR, 8 * EMBED)

    @pl.when(pl.program_id(0) == 0)
    def _():
        s1_ref[...] = jnp.zeros_like(s1_ref)
        s2_ref[...] = jnp.zeros_like(s2_ref)

    rid = lax.broadcasted_iota(jnp.int32, (BR, 1), 0)
    real = (pl.program_id(0) * BR + rid) < NREAL
    ym = jnp.where(real, y, 0.0)
    s1_ref[...] += jnp.sum(ym, axis=0, keepdims=True)
    s2_ref[...] += jnp.sum(ym * ym, axis=0, keepdims=True)


def _bn2_body(y_ref, s1_ref, s2_ref, g_ref, be_ref, out_ref):
    mu = _fold8(s1_ref[...], EMBED) / N
    var = _fold8(s2_ref[...], EMBED) / N - mu * mu
    inv = lax.rsqrt(var + 1e-5)
    mu8 = jnp.concatenate([mu] * 8, axis=1)
    inv8 = jnp.concatenate([inv] * 8, axis=1)
    out = g_ref[...] * (y_ref[0] - mu8) * inv8 + be_ref[...]
    out_ref[...] = out.reshape(1, BR, 8 * EMBED)


def _full(shape):
    return pl.BlockSpec(shape, lambda i: tuple(0 for _ in shape))


def _pk3(lanes):
    return pl.BlockSpec((1, BR, lanes), lambda i: (i, 0, 0))


def _pk4(lead):
    return pl.BlockSpec((lead, 1, BR, 128), lambda i: (0, i, 0, 0))


def kernel(x, edge_index, node_type_ids, mask_faculty, mask_course,
           mask_section, mask_room, mask_timeslot, W_faculty, b_faculty,
           W_course, b_course, W_section, b_section, W_room, b_room,
           W_timeslot, b_timeslot, sage1_Wl, sage1_Wr, sage1_b,
           sage2_Wl, sage2_Wr, sage2_b, bn1_gamma, bn1_beta,
           bn2_gamma, bn2_beta):
    f32 = jnp.float32
    eye8 = jnp.eye(8, dtype=f32)
    Ws = [W_faculty, W_course, W_section, W_room, W_timeslot]
    bs = [b_faculty, b_course, b_section, b_room, b_timeslot]
    wpk = jnp.stack([jnp.kron(eye8, jnp.pad(w, ((0, 8 - w.shape[0]),
                                                (0, 0))))
                     for w in Ws])
    bpk = jnp.tile(jnp.stack(bs), (1, 8))
    i64 = jnp.eye(HIDDEN, dtype=f32)
    epk = jnp.stack([jnp.kron(eye8, i64[:, p * 16:(p + 1) * 16])
                     for p in range(4)])
    q = jnp.arange(512)
    rowperm = (q % 128 // 16) * 64 + (q // 128) * 16 + q % 16
    wl1 = jnp.kron(eye8, sage1_Wl)[rowperm]
    wr1 = jnp.kron(eye8, sage1_Wr)
    b1 = jnp.tile(sage1_b, 8).reshape(1, 512)
    w2a = jnp.kron(eye8, sage2_Wl[:, :16])
    w2b = jnp.kron(eye8, sage2_Wl[:, 16:])
    wr2 = jnp.kron(eye8, sage2_Wr)
    b2 = jnp.tile(sage2_b, 8).reshape(1, 256)
    q2 = jnp.arange(256)
    c2col = (q2 % 128 // 16) * 32 + (q2 // 128) * 16 + q2 % 16
    p256 = jnp.zeros((256, 256), f32).at[q2, c2col].set(1.0)
    g512 = jnp.tile(bn1_gamma, 8).reshape(1, 512)
    be512 = jnp.tile(bn1_beta, 8).reshape(1, 512)
    g256 = jnp.tile(bn2_gamma, 8).reshape(1, 256)
    be256 = jnp.tile(bn2_beta, 8).reshape(1, 256)

    xpk = jnp.pad(x, ((0, NP - N), (0, 0))).reshape(GRID, BR, 64)
    tidp = jnp.pad(node_type_ids, (0, NP - N)).reshape(NP // 8, 8)
    tid512 = jnp.repeat(tidp, HIDDEN, axis=1).reshape(GRID, BR, 512)

    npad = EP - E
    pad_src = (jnp.arange(npad, dtype=jnp.int32) * 17) % N
    pad_dst = N + (jnp.arange(npad, dtype=jnp.int32) % (NP - N))
    src2d = jnp.concatenate([edge_index[0], pad_src]).reshape(RP_ALLOC, LN)
    dst2d = jnp.concatenate([edge_index[1], pad_dst]).reshape(RP_ALLOC, LN)
    z2d = jnp.zeros((NP, 16), f32)

    h, hc0, hc1, hc2, hc3 = pl.pallas_call(
        _proj_body,
        grid=(GRID,),
        in_specs=[_pk3(64), _pk3(512), _full((5, 64, 512)),
                  _full((5, 512)), _full((4, 512, 128))],
        out_specs=[_pk3(512)] + [_pk3(128)] * 4,
        out_shape=[jax.ShapeDtypeStruct((GRID, BR, 512), f32)]
        + [jax.ShapeDtypeStruct((GRID, BR, 128), f32)] * 4,
    )(xpk, tid512, wpk, bpk, epk)

    agg1, degp = _make_segsum(4, True)(
        src2d, dst2d, z2d,
        hc0.reshape(NP, 16), hc1.reshape(NP, 16),
        hc2.reshape(NP, 16), hc3.reshape(NP, 16))
    agg1 = agg1.reshape(8, GRID, BR, 128)
    degp = degp.reshape(NC, GRID, BR, 128)

    y1, s1, s2 = pl.pallas_call(
        _combine1_body,
        grid=(GRID,),
        in_specs=[_pk3(512), _pk4(8), _pk4(NC), _full((512, 512)),
                  _full((512, 512)), _full((1, 512))],
        out_specs=[_pk3(512), _full((1, 512)), _full((1, 512))],
        out_shape=[jax.ShapeDtypeStruct((GRID, BR, 512), f32),
                   jax.ShapeDtypeStruct((1, 512), f32),
                   jax.ShapeDtypeStruct((1, 512), f32)],
    )(h, agg1, degp, wl1, wr1, b1)

    h1, g0, g1 = pl.pallas_call(
        _bn1_body,
        grid=(GRID,),
        in_specs=[_pk3(512), _full((1, 512)), _full((1, 512)),
                  _full((1, 512)), _full((1, 512)),
                  _full((512, 128)), _full((512, 128))],
        out_specs=[_pk3(512), _pk3(128), _pk3(128)],
        out_shape=[jax.ShapeDtypeStruct((GRID, BR, 512), f32),
                   jax.ShapeDtypeStruct((GRID, BR, 128), f32),
                   jax.ShapeDtypeStruct((GRID, BR, 128), f32)],
    )(y1, s1, s2, g512, be512, w2a, w2b)

    (agg2,) = _make_segsum(2, False)(src2d, dst2d, z2d,
                                     g0.reshape(NP, 16),
                                     g1.reshape(NP, 16))
    agg2 = agg2.reshape(4, GRID, BR, 128)

    y2, t1, t2 = pl.pallas_call(
        _combine2_body,
        grid=(GRID,),
        in_specs=[_pk3(512), _pk4(4), _pk4(NC), _full((512, 256)),
                  _full((1, 256)), _full((256, 256))],
        out_specs=[_pk3(256), _full((1, 256)), _full((1, 256))],
        out_shape=[jax.ShapeDtypeStruct((GRID, BR, 256), f32),
                   jax.ShapeDtypeStruct((1, 256), f32),
                   jax.ShapeDtypeStruct((1, 256), f32)],
    )(h1, agg2, degp, wr2, b2, p256)

    outp = pl.pallas_call(
        _bn2_body,
        grid=(GRID,),
        in_specs=[_pk3(256), _full((1, 256)), _full((1, 256)),
                  _full((1, 256)), _full((1, 256))],
        out_specs=_pk3(256),
        out_shape=jax.ShapeDtypeStruct((GRID, BR, 256), f32),
    )(y2, t1, t2, g256, be256)

    return outp.reshape(NP, EMBED)[:N]


# speedup vs baseline: 13.0579x; 1.0938x over previous
import functools

import jax
import jax.numpy as jnp
from jax import lax
from jax.experimental import pallas as pl
from jax.experimental.pallas import tpu as pltpu
from jax.experimental.pallas import tpu_sc as plsc

N = 100000
E = 1600000
HIDDEN = 64
EMBED = 32

LN = 128
BLK = 3
NC = 2
NT = 16
NW = NC * NT
WROWS = 396
NBLK = WROWS // BLK
PAIRS = NBLK // 2
QUADS = NBLK // 4
RP = NW * WROWS
RP_ALLOC = RP + 4 * BLK
EP = RP_ALLOC * LN

NP = 102400
NODES_PER_TILE = NP // NT
BR = 256
GRID = NP // (8 * BR)
NREAL = N // 8


def _make_segsum(nchunks, with_deg):
    out_type = [jax.ShapeDtypeStruct((NC * nchunks, NP, 16), jnp.float32)]
    if with_deg:
        out_type.append(jax.ShapeDtypeStruct((NC, NP, 16), jnp.float32))

    scratch = [
        pltpu.VMEM((BLK, LN), jnp.int32),
        pltpu.VMEM((BLK, LN), jnp.int32),
        pltpu.VMEM((BLK, LN), jnp.int32),
        pltpu.VMEM((BLK, LN), jnp.int32),
        pltpu.VMEM((BLK, LN), jnp.int32),
        pltpu.VMEM((BLK, LN), jnp.int32),
        pltpu.VMEM((BLK, LN), jnp.int32),
        pltpu.VMEM((BLK, LN), jnp.int32),
        pltpu.VMEM((BLK, LN, 16), jnp.float32),
        pltpu.VMEM((BLK, LN, 16), jnp.float32),
        pltpu.VMEM((LN, 16), jnp.float32),
        pltpu.VMEM_SHARED((NP, 16), jnp.float32),
        pltpu.SemaphoreType.DMA,
        pltpu.SemaphoreType.DMA,
        pltpu.SemaphoreType.DMA,
        pltpu.SemaphoreType.DMA,
        pltpu.SemaphoreType.DMA,
        pltpu.SemaphoreType.DMA,
        pltpu.SemaphoreType.DMA,
    ]

    mesh = plsc.VectorSubcoreMesh(core_axis_name="c", subcore_axis_name="s")

    @functools.partial(
        pl.kernel, out_type=out_type, mesh=mesh, scratch_types=scratch,
        compiler_params=pltpu.CompilerParams(use_tc_tiling_on_sc=False))
    def segsum(*refs):
        (src2d, dst2d, z2d, *tabs) = refs[:3 + nchunks]
        if with_deg:
            out, deg_out = refs[3 + nchunks:3 + nchunks + 2]
            scr = refs[3 + nchunks + 2:]
        else:
            out = refs[3 + nchunks]
            scr = refs[3 + nchunks + 1:]
        (idxA_s, idxA_d, idxB_s, idxB_d, idxC_s, idxC_d, idxD_s, idxD_d,
         rowsA, rowsB, ones_v, acc,
         semGA, semGB, semIA, semIB, semIC, semID, semS) = scr

        c = lax.axis_index("c")
        t = lax.axis_index("s")
        nslc = pl.ds(t * NODES_PER_TILE, NODES_PER_TILE)
        start = (c * NT + t) * WROWS

        def idx_start(row0, i_s, i_d, sem):
            pltpu.async_copy(src2d.at[pl.ds(row0, BLK)], i_s, sem)
            pltpu.async_copy(dst2d.at[pl.ds(row0, BLK)], i_d, sem)

        def idx_wait(i_s, i_d, sem):
            pltpu.make_async_copy(src2d.at[pl.ds(0, BLK)], i_s,
                                  sem).wait()
            pltpu.make_async_copy(dst2d.at[pl.ds(0, BLK)], i_d,
                                  sem).wait()

        def fire_g(tab, i_s, rows, sem):
            for j in range(BLK):
                pltpu.async_copy(tab.at[i_s.at[j]], rows.at[j], sem)

        def drain_g(tab, i_s, rows, sem):
            for j in range(BLK):
                pltpu.make_async_copy(tab.at[i_s.at[j]], rows.at[j],
                                      sem).wait()

        def scatter(vals, i_d):
            hs = [pltpu.async_copy(vals.at[j], acc.at[i_d.at[j]],
                                   semS, add=True) for j in range(BLK)]
            for h in hs:
                h.wait()

        def scatter_ones(i_d):
            hs = [pltpu.async_copy(ones_v, acc.at[i_d.at[j]],
                                   semS, add=True) for j in range(BLK)]
            for h in hs:
                h.wait()

        if with_deg:
            def fill_ones(i, carry):
                ones_v[i, :] = jnp.ones((16,), jnp.float32)
                return carry
            lax.fori_loop(0, LN, fill_ones, 0)

        passes = ([-1] if with_deg else []) + list(range(nchunks))
        for p in passes:
            pltpu.sync_copy(z2d.at[nslc], acc.at[nslc])
            plsc.subcore_barrier()

            if p < 0:
                pltpu.sync_copy(dst2d.at[pl.ds(start, BLK)], idxA_d)
                pltpu.async_copy(dst2d.at[pl.ds(start + BLK, BLK)],
                                 idxB_d, semIB)

                def deg_body(k, carry):
                    b0 = start + (2 * k) * BLK
                    scatter_ones(idxA_d)
                    pltpu.async_copy(
                        dst2d.at[pl.ds(b0 + 2 * BLK, BLK)], idxA_d,
                        semIA)
                    pltpu.make_async_copy(
                        dst2d.at[pl.ds(0, BLK)], idxB_d, semIB).wait()
                    scatter_ones(idxB_d)
                    pltpu.async_copy(
                        dst2d.at[pl.ds(b0 + 3 * BLK, BLK)], idxB_d,
                        semIB)
                    pltpu.make_async_copy(
                        dst2d.at[pl.ds(0, BLK)], idxA_d, semIA).wait()
                    return carry

                lax.fori_loop(0, PAIRS, deg_body, 0)
                pltpu.make_async_copy(dst2d.at[pl.ds(0, BLK)], idxB_d,
                                      semIB).wait()
            else:
                tab = tabs[p]
                pltpu.sync_copy(src2d.at[pl.ds(start, BLK)], idxA_s)
                pltpu.sync_copy(dst2d.at[pl.ds(start, BLK)], idxA_d)
                fire_g(tab, idxA_s, rowsA, semGA)
                idx_start(start + BLK, idxB_s, idxB_d, semIB)
                idx_start(start + 2 * BLK, idxC_s, idxC_d, semIC)

                def quad_body(k, carry, tab=tab):
                    b = start + (4 * k) * BLK
                    idx_wait(idxB_s, idxB_d, semIB)
                    fire_g(tab, idxB_s, rowsB, semGB)
                    idx_start(b + 3 * BLK, idxD_s, idxD_d, semID)
                    drain_g(tab, idxA_s, rowsA, semGA)
                    scatter(rowsA, idxA_d)
                    idx_start(b + 4 * BLK, idxA_s, idxA_d, semIA)
                    idx_wait(idxC_s, idxC_d, semIC)
                    fire_g(tab, idxC_s, rowsA, semGA)
                    drain_g(tab, idxB_s, rowsB, semGB)
                    scatter(rowsB, idxB_d)
                    idx_start(b + 5 * BLK, idxB_s, idxB_d, semIB)
                    idx_wait(idxD_s, idxD_d, semID)
                    fire_g(tab, idxD_s, rowsB, semGB)
                    drain_g(tab, idxC_s, rowsA, semGA)
                    scatter(rowsA, idxC_d)
                    idx_start(b + 6 * BLK, idxC_s, idxC_d, semIC)
                    idx_wait(idxA_s, idxA_d, semIA)
                    fire_g(tab, idxA_s, rowsA, semGA)
                    drain_g(tab, idxD_s, rowsB, semGB)
                    scatter(rowsB, idxD_d)
                    return carry

                lax.fori_loop(0, QUADS, quad_body, 0)
                idx_wait(idxB_s, idxB_d, semIB)
                idx_wait(idxC_s, idxC_d, semIC)
                drain_g(tab, idxA_s, rowsA, semGA)

            plsc.subcore_barrier()
            if p < 0:
                pltpu.sync_copy(acc.at[nslc], deg_out.at[c, nslc])
            else:
                pltpu.sync_copy(acc.at[nslc],
                                out.at[c * nchunks + p, nslc])

    return segsum


def _proj_body(x_ref, tid_ref, w_ref, b_ref, ep_ref, k_ref,
               h_ref, c0, c1, c2, c3):
    xb = x_ref[0]
    tid = tid_ref[0]
    acc = jnp.zeros((BR, 8 * HIDDEN), jnp.float32)
    for t in range(5):
        z = jnp.dot(xb, w_ref[t], preferred_element_type=jnp.float32)
        z = jnp.maximum(z + b_ref[t], 0.0)
        mt = jnp.dot((tid == t).astype(jnp.float32), k_ref[...],
                     preferred_element_type=jnp.float32)
        acc = acc + mt * z
    h_ref[...] = acc.reshape(1, BR, 8 * HIDDEN)
    outs = [c0, c1, c2, c3]
    for p in range(4):
        cp = jnp.dot(acc, ep_ref[p], preferred_element_type=jnp.float32)
        outs[p][...] = cp.reshape(1, BR, 128)


def _combine1_body(h_ref, agg_ref, deg_ref, wl_ref, wr_ref, b_ref,
                   y_ref, s1_ref, s2_ref):
    a = agg_ref[...]
    d = jnp.maximum(deg_ref[0, 0] + deg_ref[1, 0], 1.0)
    mp = [(a[p, 0] + a[4 + p, 0]) / d for p in range(4)]
    mcat = jnp.concatenate(mp, axis=1)
    y = (jnp.dot(mcat, wl_ref[...], preferred_element_type=jnp.float32)
         + jnp.dot(h_ref[0], wr_ref[...],
                   preferred_element_type=jnp.float32)
         + b_ref[...])
    y_ref[...] = y.reshape(1, BR, 8 * HIDDEN)

    @pl.when(pl.program_id(0) == 0)
    def _():
        s1_ref[...] = jnp.zeros_like(s1_ref)
        s2_ref[...] = jnp.zeros_like(s2_ref)

    rid = lax.broadcasted_iota(jnp.int32, (BR, 1), 0)
    real = (pl.program_id(0) * BR + rid) < NREAL
    ym = jnp.where(real, y, 0.0)
    s1_ref[...] += jnp.sum(ym, axis=0, keepdims=True)
    s2_ref[...] += jnp.sum(ym * ym, axis=0, keepdims=True)


def _fold8(s, width):
    parts = [s[:, k * width:(k + 1) * width] for k in range(8)]
    tot = parts[0]
    for q in parts[1:]:
        tot = tot + q
    return tot


def _bn1_body(y_ref, s1_ref, s2_ref, g_ref, be_ref, w2a_ref, w2b_ref,
              h1_ref, g0_ref, g1_ref):
    mu = _fold8(s1_ref[...], HIDDEN) / N
    var = _fold8(s2_ref[...], HIDDEN) / N - mu * mu
    inv = lax.rsqrt(var + 1e-5)
    mu8 = jnp.concatenate([mu] * 8, axis=1)
    inv8 = jnp.concatenate([inv] * 8, axis=1)
    h1 = g_ref[...] * (y_ref[0] - mu8) * inv8 + be_ref[...]
    h1 = jnp.maximum(h1, 0.0)
    h1_ref[...] = h1.reshape(1, BR, 8 * HIDDEN)
    g0 = jnp.dot(h1, w2a_ref[...], preferred_element_type=jnp.float32)
    g1 = jnp.dot(h1, w2b_ref[...], preferred_element_type=jnp.float32)
    g0_ref[...] = g0.reshape(1, BR, 128)
    g1_ref[...] = g1.reshape(1, BR, 128)


def _combine2_body(h1_ref, agg_ref, deg_ref, wr2_ref, b2_ref, p_ref,
                   y_ref, s1_ref, s2_ref):
    a = agg_ref[...]
    d = jnp.maximum(deg_ref[0, 0] + deg_ref[1, 0], 1.0)
    mp = [(a[p, 0] + a[2 + p, 0]) / d for p in range(2)]
    mcat = jnp.concatenate(mp, axis=1)
    y = (jnp.dot(mcat, p_ref[...], preferred_element_type=jnp.float32)
         + jnp.dot(h1_ref[0], wr2_ref[...],
                   preferred_element_type=jnp.float32)
         + b2_ref[...])
    y_ref[...] = y.reshape(1, BR, 8 * EMBED)

    @pl.when(pl.program_id(0) == 0)
    def _():
        s1_ref[...] = jnp.zeros_like(s1_ref)
        s2_ref[...] = jnp.zeros_like(s2_ref)

    rid = lax.broadcasted_iota(jnp.int32, (BR, 1), 0)
    real = (pl.program_id(0) * BR + rid) < NREAL
    ym = jnp.where(real, y, 0.0)
    s1_ref[...] += jnp.sum(ym, axis=0, keepdims=True)
    s2_ref[...] += jnp.sum(ym * ym, axis=0, keepdims=True)


def _bn2_body(y_ref, s1_ref, s2_ref, g_ref, be_ref, out_ref):
    mu = _fold8(s1_ref[...], EMBED) / N
    var = _fold8(s2_ref[...], EMBED) / N - mu * mu
    inv = lax.rsqrt(var + 1e-5)
    mu8 = jnp.concatenate([mu] * 8, axis=1)
    inv8 = jnp.concatenate([inv] * 8, axis=1)
    out = g_ref[...] * (y_ref[0] - mu8) * inv8 + be_ref[...]
    out_ref[...] = out.reshape(1, BR, 8 * EMBED)


def _full(shape):
    return pl.BlockSpec(shape, lambda i: tuple(0 for _ in shape))


def _pk3(lanes):
    return pl.BlockSpec((1, BR, lanes), lambda i: (i, 0, 0))


def _pk4(lead):
    return pl.BlockSpec((lead, 1, BR, 128), lambda i: (0, i, 0, 0))


def kernel(x, edge_index, node_type_ids, mask_faculty, mask_course,
           mask_section, mask_room, mask_timeslot, W_faculty, b_faculty,
           W_course, b_course, W_section, b_section, W_room, b_room,
           W_timeslot, b_timeslot, sage1_Wl, sage1_Wr, sage1_b,
           sage2_Wl, sage2_Wr, sage2_b, bn1_gamma, bn1_beta,
           bn2_gamma, bn2_beta):
    f32 = jnp.float32
    eye8 = jnp.eye(8, dtype=f32)
    Ws = [W_faculty, W_course, W_section, W_room, W_timeslot]
    bs = [b_faculty, b_course, b_section, b_room, b_timeslot]
    wpk = jnp.stack([jnp.kron(eye8, jnp.pad(w, ((0, 8 - w.shape[0]),
                                                (0, 0))))
                     for w in Ws])
    bpk = jnp.tile(jnp.stack(bs), (1, 8))
    i64 = jnp.eye(HIDDEN, dtype=f32)
    epk = jnp.stack([jnp.kron(eye8, i64[:, p * 16:(p + 1) * 16])
                     for p in range(4)])
    q = jnp.arange(512)
    rowperm = (q % 128 // 16) * 64 + (q // 128) * 16 + q % 16
    wl1 = jnp.kron(eye8, sage1_Wl)[rowperm]
    wr1 = jnp.kron(eye8, sage1_Wr)
    b1 = jnp.tile(sage1_b, 8).reshape(1, 512)
    w2a = jnp.kron(eye8, sage2_Wl[:, :16])
    w2b = jnp.kron(eye8, sage2_Wl[:, 16:])
    wr2 = jnp.kron(eye8, sage2_Wr)
    b2 = jnp.tile(sage2_b, 8).reshape(1, 256)
    q2 = jnp.arange(256)
    c2col = (q2 % 128 // 16) * 32 + (q2 // 128) * 16 + q2 % 16
    p256 = jnp.zeros((256, 256), f32).at[q2, c2col].set(1.0)
    g512 = jnp.tile(bn1_gamma, 8).reshape(1, 512)
    be512 = jnp.tile(bn1_beta, 8).reshape(1, 512)
    g256 = jnp.tile(bn2_gamma, 8).reshape(1, 256)
    be256 = jnp.tile(bn2_beta, 8).reshape(1, 256)

    xpk = jnp.pad(x, ((0, NP - N), (0, 0))).reshape(GRID, BR, 64)
    tid8 = jnp.pad(node_type_ids, (0, NP - N)).reshape(GRID, BR, 8)
    kmask = jnp.kron(eye8, jnp.ones((1, HIDDEN), f32))

    npad = EP - E
    pad_src = (jnp.arange(npad, dtype=jnp.int32) * 17) % N
    pad_dst = N + (jnp.arange(npad, dtype=jnp.int32) % (NP - N))
    src2d = jnp.concatenate([edge_index[0], pad_src]).reshape(RP_ALLOC, LN)
    dst2d = jnp.concatenate([edge_index[1], pad_dst]).reshape(RP_ALLOC, LN)
    z2d = jnp.zeros((NP, 16), f32)

    h, hc0, hc1, hc2, hc3 = pl.pallas_call(
        _proj_body,
        grid=(GRID,),
        in_specs=[_pk3(64), _pk3(8), _full((5, 64, 512)),
                  _full((5, 512)), _full((4, 512, 128)),
                  _full((8, 512))],
        out_specs=[_pk3(512)] + [_pk3(128)] * 4,
        out_shape=[jax.ShapeDtypeStruct((GRID, BR, 512), f32)]
        + [jax.ShapeDtypeStruct((GRID, BR, 128), f32)] * 4,
    )(xpk, tid8, wpk, bpk, epk, kmask)

    agg1, degp = _make_segsum(4, True)(
        src2d, dst2d, z2d,
        hc0.reshape(NP, 16), hc1.reshape(NP, 16),
        hc2.reshape(NP, 16), hc3.reshape(NP, 16))
    agg1 = agg1.reshape(8, GRID, BR, 128)
    degp = degp.reshape(NC, GRID, BR, 128)

    y1, s1, s2 = pl.pallas_call(
        _combine1_body,
        grid=(GRID,),
        in_specs=[_pk3(512), _pk4(8), _pk4(NC), _full((512, 512)),
                  _full((512, 512)), _full((1, 512))],
        out_specs=[_pk3(512), _full((1, 512)), _full((1, 512))],
        out_shape=[jax.ShapeDtypeStruct((GRID, BR, 512), f32),
                   jax.ShapeDtypeStruct((1, 512), f32),
                   jax.ShapeDtypeStruct((1, 512), f32)],
    )(h, agg1, degp, wl1, wr1, b1)

    h1, g0, g1 = pl.pallas_call(
        _bn1_body,
        grid=(GRID,),
        in_specs=[_pk3(512), _full((1, 512)), _full((1, 512)),
                  _full((1, 512)), _full((1, 512)),
                  _full((512, 128)), _full((512, 128))],
        out_specs=[_pk3(512), _pk3(128), _pk3(128)],
        out_shape=[jax.ShapeDtypeStruct((GRID, BR, 512), f32),
                   jax.ShapeDtypeStruct((GRID, BR, 128), f32),
                   jax.ShapeDtypeStruct((GRID, BR, 128), f32)],
    )(y1, s1, s2, g512, be512, w2a, w2b)

    (agg2,) = _make_segsum(2, False)(src2d, dst2d, z2d,
                                     g0.reshape(NP, 16),
                                     g1.reshape(NP, 16))
    agg2 = agg2.reshape(4, GRID, BR, 128)

    y2, t1, t2 = pl.pallas_call(
        _combine2_body,
        grid=(GRID,),
        in_specs=[_pk3(512), _pk4(4), _pk4(NC), _full((512, 256)),
                  _full((1, 256)), _full((256, 256))],
        out_specs=[_pk3(256), _full((1, 256)), _full((1, 256))],
        out_shape=[jax.ShapeDtypeStruct((GRID, BR, 256), f32),
                   jax.ShapeDtypeStruct((1, 256), f32),
                   jax.ShapeDtypeStruct((1, 256), f32)],
    )(h1, agg2, degp, wr2, b2, p256)

    outp = pl.pallas_call(
        _bn2_body,
        grid=(GRID,),
        in_specs=[_pk3(256), _full((1, 256)), _full((1, 256)),
                  _full((1, 256)), _full((1, 256))],
        out_specs=_pk3(256),
        out_shape=jax.ShapeDtypeStruct((GRID, BR, 256), f32),
    )(y2, t1, t2, g256, be256)

    return outp.reshape(NP // 8, 8 * EMBED)[:NREAL].reshape(N, EMBED)


# speedup vs baseline: 13.6628x; 1.0463x over previous
import functools

import numpy as np

import jax
import jax.numpy as jnp
from jax import lax
from jax.experimental import pallas as pl
from jax.experimental.pallas import tpu as pltpu
from jax.experimental.pallas import tpu_sc as plsc

N = 100000
E = 1600000
HIDDEN = 64
EMBED = 32

LN = 128
BLK = 3
NC = 2
NT = 16
NW = NC * NT
WROWS = 396
NBLK = WROWS // BLK
PAIRS = NBLK // 2
QUADS = NBLK // 4
RP = NW * WROWS
RP_ALLOC = RP + 4 * BLK
EP = RP_ALLOC * LN

NP = 102400
NODES_PER_TILE = NP // NT
BR = 512
GRID = NP // (8 * BR)
NREAL = N // 8


def _make_segsum(nchunks, with_deg):
    out_type = [jax.ShapeDtypeStruct((NC * nchunks, NP, 16), jnp.float32)]
    if with_deg:
        out_type.append(jax.ShapeDtypeStruct((NC, NP, 16), jnp.float32))

    scratch = [
        pltpu.VMEM((BLK, LN), jnp.int32),
        pltpu.VMEM((BLK, LN), jnp.int32),
        pltpu.VMEM((BLK, LN), jnp.int32),
        pltpu.VMEM((BLK, LN), jnp.int32),
        pltpu.VMEM((BLK, LN), jnp.int32),
        pltpu.VMEM((BLK, LN), jnp.int32),
        pltpu.VMEM((BLK, LN), jnp.int32),
        pltpu.VMEM((BLK, LN), jnp.int32),
        pltpu.VMEM((BLK, LN, 16), jnp.float32),
        pltpu.VMEM((BLK, LN, 16), jnp.float32),
        pltpu.VMEM((LN, 16), jnp.float32),
        pltpu.VMEM_SHARED((NP, 16), jnp.float32),
        pltpu.SemaphoreType.DMA,
        pltpu.SemaphoreType.DMA,
        pltpu.SemaphoreType.DMA,
        pltpu.SemaphoreType.DMA,
        pltpu.SemaphoreType.DMA,
        pltpu.SemaphoreType.DMA,
        pltpu.SemaphoreType.DMA,
    ]

    mesh = plsc.VectorSubcoreMesh(core_axis_name="c", subcore_axis_name="s")

    @functools.partial(
        pl.kernel, out_type=out_type, mesh=mesh, scratch_types=scratch,
        compiler_params=pltpu.CompilerParams(use_tc_tiling_on_sc=False))
    def segsum(*refs):
        (src2d, dst2d, z2d, *tabs) = refs[:3 + nchunks]
        if with_deg:
            out, deg_out = refs[3 + nchunks:3 + nchunks + 2]
            scr = refs[3 + nchunks + 2:]
        else:
            out = refs[3 + nchunks]
            scr = refs[3 + nchunks + 1:]
        (idxA_s, idxA_d, idxB_s, idxB_d, idxC_s, idxC_d, idxD_s, idxD_d,
         rowsA, rowsB, ones_v, acc,
         semGA, semGB, semIA, semIB, semIC, semID, semS) = scr

        c = lax.axis_index("c")
        t = lax.axis_index("s")
        nslc = pl.ds(t * NODES_PER_TILE, NODES_PER_TILE)
        start = (c * NT + t) * WROWS

        def idx_start(row0, i_s, i_d, sem):
            pltpu.async_copy(src2d.at[pl.ds(row0, BLK)], i_s, sem)
            pltpu.async_copy(dst2d.at[pl.ds(row0, BLK)], i_d, sem)

        def idx_wait(i_s, i_d, sem):
            pltpu.make_async_copy(src2d.at[pl.ds(0, BLK)], i_s,
                                  sem).wait()
            pltpu.make_async_copy(dst2d.at[pl.ds(0, BLK)], i_d,
                                  sem).wait()

        def fire_g(tab, i_s, rows, sem):
            for j in range(BLK):
                pltpu.async_copy(tab.at[i_s.at[j]], rows.at[j], sem)

        def drain_g(tab, i_s, rows, sem):
            for j in range(BLK):
                pltpu.make_async_copy(tab.at[i_s.at[j]], rows.at[j],
                                      sem).wait()

        def scatter(vals, i_d):
            hs = [pltpu.async_copy(vals.at[j], acc.at[i_d.at[j]],
                                   semS, add=True) for j in range(BLK)]
            for h in hs:
                h.wait()

        def scatter_ones(i_d):
            hs = [pltpu.async_copy(ones_v, acc.at[i_d.at[j]],
                                   semS, add=True) for j in range(BLK)]
            for h in hs:
                h.wait()

        if with_deg:
            def fill_ones(i, carry):
                ones_v[i, :] = jnp.ones((16,), jnp.float32)
                return carry
            lax.fori_loop(0, LN, fill_ones, 0)

        passes = ([-1] if with_deg else []) + list(range(nchunks))
        for p in passes:
            pltpu.sync_copy(z2d.at[nslc], acc.at[nslc])
            plsc.subcore_barrier()

            if p < 0:
                pltpu.sync_copy(dst2d.at[pl.ds(start, BLK)], idxA_d)
                pltpu.async_copy(dst2d.at[pl.ds(start + BLK, BLK)],
                                 idxB_d, semIB)

                def deg_body(k, carry):
                    b0 = start + (2 * k) * BLK
                    scatter_ones(idxA_d)
                    pltpu.async_copy(
                        dst2d.at[pl.ds(b0 + 2 * BLK, BLK)], idxA_d,
                        semIA)
                    pltpu.make_async_copy(
                        dst2d.at[pl.ds(0, BLK)], idxB_d, semIB).wait()
                    scatter_ones(idxB_d)
                    pltpu.async_copy(
                        dst2d.at[pl.ds(b0 + 3 * BLK, BLK)], idxB_d,
                        semIB)
                    pltpu.make_async_copy(
                        dst2d.at[pl.ds(0, BLK)], idxA_d, semIA).wait()
                    return carry

                lax.fori_loop(0, PAIRS, deg_body, 0)
                pltpu.make_async_copy(dst2d.at[pl.ds(0, BLK)], idxB_d,
                                      semIB).wait()
            else:
                tab = tabs[p]
                pltpu.sync_copy(src2d.at[pl.ds(start, BLK)], idxA_s)
                pltpu.sync_copy(dst2d.at[pl.ds(start, BLK)], idxA_d)
                fire_g(tab, idxA_s, rowsA, semGA)
                idx_start(start + BLK, idxB_s, idxB_d, semIB)
                idx_start(start + 2 * BLK, idxC_s, idxC_d, semIC)

                def quad_body(k, carry, tab=tab):
                    b = start + (4 * k) * BLK
                    idx_wait(idxB_s, idxB_d, semIB)
                    fire_g(tab, idxB_s, rowsB, semGB)
                    idx_start(b + 3 * BLK, idxD_s, idxD_d, semID)
                    drain_g(tab, idxA_s, rowsA, semGA)
                    scatter(rowsA, idxA_d)
                    idx_start(b + 4 * BLK, idxA_s, idxA_d, semIA)
                    idx_wait(idxC_s, idxC_d, semIC)
                    fire_g(tab, idxC_s, rowsA, semGA)
                    drain_g(tab, idxB_s, rowsB, semGB)
                    scatter(rowsB, idxB_d)
                    idx_start(b + 5 * BLK, idxB_s, idxB_d, semIB)
                    idx_wait(idxD_s, idxD_d, semID)
                    fire_g(tab, idxD_s, rowsB, semGB)
                    drain_g(tab, idxC_s, rowsA, semGA)
                    scatter(rowsA, idxC_d)
                    idx_start(b + 6 * BLK, idxC_s, idxC_d, semIC)
                    idx_wait(idxA_s, idxA_d, semIA)
                    fire_g(tab, idxA_s, rowsA, semGA)
                    drain_g(tab, idxD_s, rowsB, semGB)
                    scatter(rowsB, idxD_d)
                    return carry

                lax.fori_loop(0, QUADS, quad_body, 0)
                idx_wait(idxB_s, idxB_d, semIB)
                idx_wait(idxC_s, idxC_d, semIC)
                drain_g(tab, idxA_s, rowsA, semGA)

            plsc.subcore_barrier()
            if p < 0:
                pltpu.sync_copy(acc.at[nslc], deg_out.at[c, nslc])
            else:
                pltpu.sync_copy(acc.at[nslc],
                                out.at[c * nchunks + p, nslc])

    return segsum


def _packed_h(xb, w_ref, b_ref, t_ref, k_ref):
    tidf = jnp.dot(xb, t_ref[...], preferred_element_type=jnp.float32)
    acc = jnp.zeros((BR, 8 * HIDDEN), jnp.float32)
    for t in range(5):
        z = jnp.dot(xb, w_ref[t], preferred_element_type=jnp.float32)
        z = jnp.maximum(z + b_ref[t], 0.0)
        mt = jnp.dot((tidf == float(t)).astype(jnp.float32), k_ref[...],
                     preferred_element_type=jnp.float32)
        acc = acc + mt * z
    return acc


def _proj_body(x_ref, w_ref, b_ref, t_ref, k_ref, ep_ref,
               c0, c1, c2, c3):
    acc = _packed_h(x_ref[0], w_ref, b_ref, t_ref, k_ref)
    outs = [c0, c1, c2, c3]
    for p in range(4):
        cp = jnp.dot(acc, ep_ref[p], preferred_element_type=jnp.float32)
        outs[p][...] = cp.reshape(1, BR, 128)


def _combine1_body(x_ref, w_ref, b_ref, t_ref, k_ref,
                   agg_ref, deg_ref, wl_ref, wr_ref, b1_ref,
                   y_ref, s1_ref, s2_ref):
    h = _packed_h(x_ref[0], w_ref, b_ref, t_ref, k_ref)
    a = agg_ref[...]
    d = jnp.maximum(deg_ref[0, 0] + deg_ref[1, 0], 1.0)
    mp = [(a[p, 0] + a[4 + p, 0]) / d for p in range(4)]
    mcat = jnp.concatenate(mp, axis=1)
    y = (jnp.dot(mcat, wl_ref[...], preferred_element_type=jnp.float32)
         + jnp.dot(h, wr_ref[...], preferred_element_type=jnp.float32)
         + b1_ref[...])
    y_ref[...] = y.reshape(1, BR, 8 * HIDDEN)

    @pl.when(pl.program_id(0) == 0)
    def _():
        s1_ref[...] = jnp.zeros_like(s1_ref)
        s2_ref[...] = jnp.zeros_like(s2_ref)

    rid = lax.broadcasted_iota(jnp.int32, (BR, 1), 0)
    real = (pl.program_id(0) * BR + rid) < NREAL
    ym = jnp.where(real, y, 0.0)
    s1_ref[...] += jnp.sum(ym, axis=0, keepdims=True)
    s2_ref[...] += jnp.sum(ym * ym, axis=0, keepdims=True)


def _fold8(s, width):
    parts = [s[:, k * width:(k + 1) * width] for k in range(8)]
    tot = parts[0]
    for q in parts[1:]:
        tot = tot + q
    return tot


def _bn1_body(y_ref, s1_ref, s2_ref, g_ref, be_ref, w2a_ref, w2b_ref,
              h1_ref, g0_ref, g1_ref):
    mu = _fold8(s1_ref[...], HIDDEN) / N
    var = _fold8(s2_ref[...], HIDDEN) / N - mu * mu
    inv = lax.rsqrt(var + 1e-5)
    mu8 = jnp.concatenate([mu] * 8, axis=1)
    inv8 = jnp.concatenate([inv] * 8, axis=1)
    h1 = g_ref[...] * (y_ref[0] - mu8) * inv8 + be_ref[...]
    h1 = jnp.maximum(h1, 0.0)
    h1_ref[...] = h1.reshape(1, BR, 8 * HIDDEN)
    g0 = jnp.dot(h1, w2a_ref[...], preferred_element_type=jnp.float32)
    g1 = jnp.dot(h1, w2b_ref[...], preferred_element_type=jnp.float32)
    g0_ref[...] = g0.reshape(1, BR, 128)
    g1_ref[...] = g1.reshape(1, BR, 128)


def _combine2_body(h1_ref, agg_ref, deg_ref, wr2_ref, b2_ref, p_ref,
                   y_ref, s1_ref, s2_ref):
    a = agg_ref[...]
    d = jnp.maximum(deg_ref[0, 0] + deg_ref[1, 0], 1.0)
    mp = [(a[p, 0] + a[2 + p, 0]) / d for p in range(2)]
    mcat = jnp.concatenate(mp, axis=1)
    y = (jnp.dot(mcat, p_ref[...], preferred_element_type=jnp.float32)
         + jnp.dot(h1_ref[0], wr2_ref[...],
                   preferred_element_type=jnp.float32)
         + b2_ref[...])
    y_ref[...] = y.reshape(1, BR, 8 * EMBED)

    @pl.when(pl.program_id(0) == 0)
    def _():
        s1_ref[...] = jnp.zeros_like(s1_ref)
        s2_ref[...] = jnp.zeros_like(s2_ref)

    rid = lax.broadcasted_iota(jnp.int32, (BR, 1), 0)
    real = (pl.program_id(0) * BR + rid) < NREAL
    ym = jnp.where(real, y, 0.0)
    s1_ref[...] += jnp.sum(ym, axis=0, keepdims=True)
    s2_ref[...] += jnp.sum(ym * ym, axis=0, keepdims=True)


def _bn2_body(y_ref, s1_ref, s2_ref, g_ref, be_ref, out_ref):
    mu = _fold8(s1_ref[...], EMBED) / N
    var = _fold8(s2_ref[...], EMBED) / N - mu * mu
    inv = lax.rsqrt(var + 1e-5)
    mu8 = jnp.concatenate([mu] * 8, axis=1)
    inv8 = jnp.concatenate([inv] * 8, axis=1)
    out = g_ref[...] * (y_ref[0] - mu8) * inv8 + be_ref[...]
    out_ref[...] = out.reshape(1, BR, 8 * EMBED)


def _full(shape):
    return pl.BlockSpec(shape, lambda i: tuple(0 for _ in shape))


def _pk3(lanes):
    return pl.BlockSpec((1, BR, lanes), lambda i: (i, 0, 0))


def _pk4(lead):
    return pl.BlockSpec((lead, 1, BR, 128), lambda i: (0, i, 0, 0))


def kernel(x, edge_index, node_type_ids, mask_faculty, mask_course,
           mask_section, mask_room, mask_timeslot, W_faculty, b_faculty,
           W_course, b_course, W_section, b_section, W_room, b_room,
           W_timeslot, b_timeslot, sage1_Wl, sage1_Wr, sage1_b,
           sage2_Wl, sage2_Wr, sage2_b, bn1_gamma, bn1_beta,
           bn2_gamma, bn2_beta):
    f32 = jnp.float32
    eye8 = np.eye(8, dtype=np.float32)
    Ws = [W_faculty, W_course, W_section, W_room, W_timeslot]
    bs = [b_faculty, b_course, b_section, b_room, b_timeslot]
    wpk = jnp.stack([jnp.kron(eye8, jnp.pad(w, ((0, 9 - w.shape[0]),
                                                (0, 0))))
                     for w in Ws])
    bpk = jnp.tile(jnp.stack(bs), (1, 8))
    i64 = np.eye(HIDDEN, dtype=np.float32)
    epk = jnp.asarray(np.stack(
        [np.kron(eye8, i64[:, p * 16:(p + 1) * 16]) for p in range(4)]))
    tmat_np = np.zeros((72, 8), np.float32)
    tmat_np[np.arange(8) * 9 + 8, np.arange(8)] = 1.0
    tmat = jnp.asarray(tmat_np)
    kmask = jnp.asarray(np.kron(eye8, np.ones((1, HIDDEN), np.float32)))
    q2 = np.arange(256)
    c2col = (q2 % 128 // 16) * 32 + (q2 // 128) * 16 + q2 % 16
    p256_np = np.zeros((256, 256), np.float32)
    p256_np[q2, c2col] = 1.0
    p256 = jnp.asarray(p256_np)
    q = jnp.arange(512)
    rowperm = (q % 128 // 16) * 64 + (q // 128) * 16 + q % 16
    wl1 = jnp.kron(eye8, sage1_Wl)[rowperm]
    wr1 = jnp.kron(eye8, sage1_Wr)
    b1 = jnp.tile(sage1_b, 8).reshape(1, 512)
    w2a = jnp.kron(eye8, sage2_Wl[:, :16])
    w2b = jnp.kron(eye8, sage2_Wl[:, 16:])
    wr2 = jnp.kron(eye8, sage2_Wr)
    b2 = jnp.tile(sage2_b, 8).reshape(1, 256)
    g512 = jnp.tile(bn1_gamma, 8).reshape(1, 512)
    be512 = jnp.tile(bn1_beta, 8).reshape(1, 512)
    g256 = jnp.tile(bn2_gamma, 8).reshape(1, 256)
    be256 = jnp.tile(bn2_beta, 8).reshape(1, 256)

    xt = jnp.concatenate(
        [x, node_type_ids.astype(f32).reshape(N, 1)], axis=1)
    xtp = jnp.pad(xt, ((0, NP - N), (0, 0))).reshape(GRID, BR, 72)

    npad = EP - E
    pad_src = (jnp.arange(npad, dtype=jnp.int32) * 17) % N
    pad_dst = N + (jnp.arange(npad, dtype=jnp.int32) % (NP - N))
    src2d = jnp.concatenate([edge_index[0], pad_src]).reshape(RP_ALLOC, LN)
    dst2d = jnp.concatenate([edge_index[1], pad_dst]).reshape(RP_ALLOC, LN)
    z2d = jnp.zeros((NP, 16), f32)

    hc0, hc1, hc2, hc3 = pl.pallas_call(
        _proj_body,
        grid=(GRID,),
        in_specs=[_pk3(72), _full((5, 72, 512)), _full((5, 512)),
                  _full((72, 8)), _full((8, 512)),
                  _full((4, 512, 128))],
        out_specs=[_pk3(128)] * 4,
        out_shape=[jax.ShapeDtypeStruct((GRID, BR, 128), f32)] * 4,
    )(xtp, wpk, bpk, tmat, kmask, epk)

    agg1, degp = _make_segsum(4, True)(
        src2d, dst2d, z2d,
        hc0.reshape(NP, 16), hc1.reshape(NP, 16),
        hc2.reshape(NP, 16), hc3.reshape(NP, 16))
    agg1 = agg1.reshape(8, GRID, BR, 128)
    degp = degp.reshape(NC, GRID, BR, 128)

    y1, s1, s2 = pl.pallas_call(
        _combine1_body,
        grid=(GRID,),
        in_specs=[_pk3(72), _full((5, 72, 512)), _full((5, 512)),
                  _full((72, 8)), _full((8, 512)),
                  _pk4(8), _pk4(NC), _full((512, 512)),
                  _full((512, 512)), _full((1, 512))],
        out_specs=[_pk3(512), _full((1, 512)), _full((1, 512))],
        out_shape=[jax.ShapeDtypeStruct((GRID, BR, 512), f32),
                   jax.ShapeDtypeStruct((1, 512), f32),
                   jax.ShapeDtypeStruct((1, 512), f32)],
    )(xtp, wpk, bpk, tmat, kmask, agg1, degp, wl1, wr1, b1)

    h1, g0, g1 = pl.pallas_call(
        _bn1_body,
        grid=(GRID,),
        in_specs=[_pk3(512), _full((1, 512)), _full((1, 512)),
                  _full((1, 512)), _full((1, 512)),
                  _full((512, 128)), _full((512, 128))],
        out_specs=[_pk3(512), _pk3(128), _pk3(128)],
        out_shape=[jax.ShapeDtypeStruct((GRID, BR, 512), f32),
                   jax.ShapeDtypeStruct((GRID, BR, 128), f32),
                   jax.ShapeDtypeStruct((GRID, BR, 128), f32)],
    )(y1, s1, s2, g512, be512, w2a, w2b)

    (agg2,) = _make_segsum(2, False)(src2d, dst2d, z2d,
                                     g0.reshape(NP, 16),
                                     g1.reshape(NP, 16))
    agg2 = agg2.reshape(4, GRID, BR, 128)

    y2, t1, t2 = pl.pallas_call(
        _combine2_body,
        grid=(GRID,),
        in_specs=[_pk3(512), _pk4(4), _pk4(NC), _full((512, 256)),
                  _full((1, 256)), _full((256, 256))],
        out_specs=[_pk3(256), _full((1, 256)), _full((1, 256))],
        out_shape=[jax.ShapeDtypeStruct((GRID, BR, 256), f32),
                   jax.ShapeDtypeStruct((1, 256), f32),
                   jax.ShapeDtypeStruct((1, 256), f32)],
    )(h1, agg2, degp, wr2, b2, p256)

    outp = pl.pallas_call(
        _bn2_body,
        grid=(GRID,),
        in_specs=[_pk3(256), _full((1, 256)), _full((1, 256)),
                  _full((1, 256)), _full((1, 256))],
        out_specs=_pk3(256),
        out_shape=jax.ShapeDtypeStruct((GRID, BR, 256), f32),
    )(y2, t1, t2, g256, be256)

    return outp.reshape(NP // 8, 8 * EMBED)[:NREAL].reshape(N, EMBED)


# speedup vs baseline: 13.7298x; 1.0049x over previous
import functools

import numpy as np

import jax
import jax.numpy as jnp
from jax import lax
from jax.experimental import pallas as pl
from jax.experimental.pallas import tpu as pltpu
from jax.experimental.pallas import tpu_sc as plsc

N = 100000
E = 1600000
HIDDEN = 64
EMBED = 32

LN = 128
BLK = 3
NC = 2
NT = 16
NW = NC * NT
WROWS = 396
NBLK = WROWS // BLK
PAIRS = NBLK // 2
QUADS = NBLK // 4
RP = NW * WROWS
RP_ALLOC = RP + 4 * BLK
EP = RP_ALLOC * LN

NP = 102400
NODES_PER_TILE = NP // NT
BR = 512
GRID = NP // (8 * BR)
NREAL = N // 8


def _make_segsum(nchunks, with_deg):
    out_type = [jax.ShapeDtypeStruct((NC * nchunks, NP, 16), jnp.float32)]
    if with_deg:
        out_type.append(jax.ShapeDtypeStruct((NC, NP, 16), jnp.float32))

    scratch = [
        pltpu.VMEM((BLK, LN), jnp.int32),
        pltpu.VMEM((BLK, LN), jnp.int32),
        pltpu.VMEM((BLK, LN), jnp.int32),
        pltpu.VMEM((BLK, LN), jnp.int32),
        pltpu.VMEM((BLK, LN), jnp.int32),
        pltpu.VMEM((BLK, LN), jnp.int32),
        pltpu.VMEM((BLK, LN), jnp.int32),
        pltpu.VMEM((BLK, LN), jnp.int32),
        pltpu.VMEM((BLK, LN, 16), jnp.float32),
        pltpu.VMEM((BLK, LN, 16), jnp.float32),
        pltpu.VMEM((LN, 16), jnp.float32),
        pltpu.VMEM_SHARED((NP, 16), jnp.float32),
        pltpu.SemaphoreType.DMA,
        pltpu.SemaphoreType.DMA,
        pltpu.SemaphoreType.DMA,
        pltpu.SemaphoreType.DMA,
        pltpu.SemaphoreType.DMA,
        pltpu.SemaphoreType.DMA,
        pltpu.SemaphoreType.DMA,
    ]

    mesh = plsc.VectorSubcoreMesh(core_axis_name="c", subcore_axis_name="s")

    @functools.partial(
        pl.kernel, out_type=out_type, mesh=mesh, scratch_types=scratch,
        compiler_params=pltpu.CompilerParams(use_tc_tiling_on_sc=False))
    def segsum(*refs):
        (src2d, dst2d, z2d, *tabs) = refs[:3 + nchunks]
        if with_deg:
            out, deg_out = refs[3 + nchunks:3 + nchunks + 2]
            scr = refs[3 + nchunks + 2:]
        else:
            out = refs[3 + nchunks]
            scr = refs[3 + nchunks + 1:]
        (idxA_s, idxA_d, idxB_s, idxB_d, idxC_s, idxC_d, idxD_s, idxD_d,
         rowsA, rowsB, ones_v, acc,
         semGA, semGB, semIA, semIB, semIC, semID, semS) = scr

        c = lax.axis_index("c")
        t = lax.axis_index("s")
        nslc = pl.ds(t * NODES_PER_TILE, NODES_PER_TILE)
        start = (c * NT + t) * WROWS

        def idx_start(row0, i_s, i_d, sem):
            pltpu.async_copy(src2d.at[pl.ds(row0, BLK)], i_s, sem)
            pltpu.async_copy(dst2d.at[pl.ds(row0, BLK)], i_d, sem)

        def idx_wait(i_s, i_d, sem):
            pltpu.make_async_copy(src2d.at[pl.ds(0, BLK)], i_s,
                                  sem).wait()
            pltpu.make_async_copy(dst2d.at[pl.ds(0, BLK)], i_d,
                                  sem).wait()

        def fire_g(tab, i_s, rows, sem):
            for j in range(BLK):
                pltpu.async_copy(tab.at[i_s.at[j]], rows.at[j], sem)

        def drain_g(tab, i_s, rows, sem):
            for j in range(BLK):
                pltpu.make_async_copy(tab.at[i_s.at[j]], rows.at[j],
                                      sem).wait()

        def scatter(vals, i_d):
            hs = [pltpu.async_copy(vals.at[j], acc.at[i_d.at[j]],
                                   semS, add=True) for j in range(BLK)]
            for h in hs:
                h.wait()

        def scatter_ones(i_d):
            hs = [pltpu.async_copy(ones_v, acc.at[i_d.at[j]],
                                   semS, add=True) for j in range(BLK)]
            for h in hs:
                h.wait()

        if with_deg:
            def fill_ones(i, carry):
                ones_v[i, :] = jnp.ones((16,), jnp.float32)
                return carry
            lax.fori_loop(0, LN, fill_ones, 0)

        passes = ([-1] if with_deg else []) + list(range(nchunks))
        for p in passes:
            pltpu.sync_copy(z2d.at[nslc], acc.at[nslc])
            plsc.subcore_barrier()

            if p < 0:
                pltpu.sync_copy(dst2d.at[pl.ds(start, BLK)], idxA_d)
                pltpu.async_copy(dst2d.at[pl.ds(start + BLK, BLK)],
                                 idxB_d, semIB)

                def deg_body(k, carry):
                    b0 = start + (2 * k) * BLK
                    scatter_ones(idxA_d)
                    pltpu.async_copy(
                        dst2d.at[pl.ds(b0 + 2 * BLK, BLK)], idxA_d,
                        semIA)
                    pltpu.make_async_copy(
                        dst2d.at[pl.ds(0, BLK)], idxB_d, semIB).wait()
                    scatter_ones(idxB_d)
                    pltpu.async_copy(
                        dst2d.at[pl.ds(b0 + 3 * BLK, BLK)], idxB_d,
                        semIB)
                    pltpu.make_async_copy(
                        dst2d.at[pl.ds(0, BLK)], idxA_d, semIA).wait()
                    return carry

                lax.fori_loop(0, PAIRS, deg_body, 0)
                pltpu.make_async_copy(dst2d.at[pl.ds(0, BLK)], idxB_d,
                                      semIB).wait()
            else:
                tab = tabs[p]
                pltpu.sync_copy(src2d.at[pl.ds(start, BLK)], idxA_s)
                pltpu.sync_copy(dst2d.at[pl.ds(start, BLK)], idxA_d)
                fire_g(tab, idxA_s, rowsA, semGA)
                idx_start(start + BLK, idxB_s, idxB_d, semIB)
                idx_start(start + 2 * BLK, idxC_s, idxC_d, semIC)

                def quad_body(k, carry, tab=tab):
                    b = start + (4 * k) * BLK
                    idx_wait(idxB_s, idxB_d, semIB)
                    fire_g(tab, idxB_s, rowsB, semGB)
                    idx_start(b + 3 * BLK, idxD_s, idxD_d, semID)
                    drain_g(tab, idxA_s, rowsA, semGA)
                    scatter(rowsA, idxA_d)
                    idx_start(b + 4 * BLK, idxA_s, idxA_d, semIA)
                    idx_wait(idxC_s, idxC_d, semIC)
                    fire_g(tab, idxC_s, rowsA, semGA)
                    drain_g(tab, idxB_s, rowsB, semGB)
                    scatter(rowsB, idxB_d)
                    idx_start(b + 5 * BLK, idxB_s, idxB_d, semIB)
                    idx_wait(idxD_s, idxD_d, semID)
                    fire_g(tab, idxD_s, rowsB, semGB)
                    drain_g(tab, idxC_s, rowsA, semGA)
                    scatter(rowsA, idxC_d)
                    idx_start(b + 6 * BLK, idxC_s, idxC_d, semIC)
                    idx_wait(idxA_s, idxA_d, semIA)
                    fire_g(tab, idxA_s, rowsA, semGA)
                    drain_g(tab, idxD_s, rowsB, semGB)
                    scatter(rowsB, idxD_d)
                    return carry

                lax.fori_loop(0, QUADS, quad_body, 0)
                idx_wait(idxB_s, idxB_d, semIB)
                idx_wait(idxC_s, idxC_d, semIC)
                drain_g(tab, idxA_s, rowsA, semGA)

            plsc.subcore_barrier()
            if p < 0:
                pltpu.sync_copy(acc.at[nslc], deg_out.at[c, nslc])
            else:
                pltpu.sync_copy(acc.at[nslc],
                                out.at[c * nchunks + p, nslc])

    return segsum


def _packed_h(xb, w_ref, b_ref, t_ref, k_ref):
    tidf = jnp.dot(xb, t_ref[...], preferred_element_type=jnp.float32)
    acc = jnp.zeros((BR, 8 * HIDDEN), jnp.float32)
    for t in range(5):
        z = jnp.dot(xb, w_ref[t], preferred_element_type=jnp.float32)
        z = jnp.maximum(z + b_ref[t], 0.0)
        mt = jnp.dot((tidf == float(t)).astype(jnp.float32), k_ref[...],
                     preferred_element_type=jnp.float32)
        acc = acc + mt * z
    return acc


def _proj_body(x_ref, w_ref, b_ref, t_ref, k_ref, ep_ref,
               c0, c1, c2, c3):
    acc = _packed_h(x_ref[0], w_ref, b_ref, t_ref, k_ref)
    outs = [c0, c1, c2, c3]
    for p in range(4):
        cp = jnp.dot(acc, ep_ref[p], preferred_element_type=jnp.float32)
        outs[p][...] = cp.reshape(1, BR, 128)


def _combine1_body(x_ref, w_ref, b_ref, t_ref, k_ref,
                   agg_ref, deg_ref, wl_ref, wr_ref, b1_ref,
                   y_ref, s1_ref, s2_ref):
    h = _packed_h(x_ref[0], w_ref, b_ref, t_ref, k_ref)
    a = agg_ref[...]
    d = jnp.maximum(deg_ref[0, 0] + deg_ref[1, 0], 1.0)
    mp = [(a[p, 0] + a[4 + p, 0]) / d for p in range(4)]
    mcat = jnp.concatenate(mp, axis=1)
    y = (jnp.dot(mcat, wl_ref[...], preferred_element_type=jnp.float32)
         + jnp.dot(h, wr_ref[...], preferred_element_type=jnp.float32)
         + b1_ref[...])
    y_ref[...] = y.reshape(1, BR, 8 * HIDDEN)

    @pl.when(pl.program_id(0) == 0)
    def _():
        s1_ref[...] = jnp.zeros_like(s1_ref)
        s2_ref[...] = jnp.zeros_like(s2_ref)

    rid = lax.broadcasted_iota(jnp.int32, (BR, 1), 0)
    real = (pl.program_id(0) * BR + rid) < NREAL
    ym = jnp.where(real, y, 0.0)
    s1_ref[...] += jnp.sum(ym, axis=0, keepdims=True)
    s2_ref[...] += jnp.sum(ym * ym, axis=0, keepdims=True)


def _fold8(s, width):
    parts = [s[:, k * width:(k + 1) * width] for k in range(8)]
    tot = parts[0]
    for q in parts[1:]:
        tot = tot + q
    return tot


def _bn1_h1(y, s1_ref, s2_ref, g_ref, be_ref):
    mu = _fold8(s1_ref[...], HIDDEN) / N
    var = _fold8(s2_ref[...], HIDDEN) / N - mu * mu
    inv = lax.rsqrt(var + 1e-5)
    mu8 = jnp.concatenate([mu] * 8, axis=1)
    inv8 = jnp.concatenate([inv] * 8, axis=1)
    h1 = g_ref[...] * (y - mu8) * inv8 + be_ref[...]
    return jnp.maximum(h1, 0.0)


def _bn1_body(y_ref, s1_ref, s2_ref, g_ref, be_ref, w2a_ref, w2b_ref,
              g0_ref, g1_ref):
    h1 = _bn1_h1(y_ref[0], s1_ref, s2_ref, g_ref, be_ref)
    g0 = jnp.dot(h1, w2a_ref[...], preferred_element_type=jnp.float32)
    g1 = jnp.dot(h1, w2b_ref[...], preferred_element_type=jnp.float32)
    g0_ref[...] = g0.reshape(1, BR, 128)
    g1_ref[...] = g1.reshape(1, BR, 128)


def _combine2_body(y1_ref, s1_ref, s2_ref, g1_ref, be1_ref,
                   agg_ref, deg_ref, wr2_ref, b2_ref, p_ref,
                   y_ref, sa_ref, sb_ref):
    h1 = _bn1_h1(y1_ref[0], s1_ref, s2_ref, g1_ref, be1_ref)
    a = agg_ref[...]
    d = jnp.maximum(deg_ref[0, 0] + deg_ref[1, 0], 1.0)
    mp = [(a[p, 0] + a[2 + p, 0]) / d for p in range(2)]
    mcat = jnp.concatenate(mp, axis=1)
    y = (jnp.dot(mcat, p_ref[...], preferred_element_type=jnp.float32)
         + jnp.dot(h1, wr2_ref[...], preferred_element_type=jnp.float32)
         + b2_ref[...])
    y_ref[...] = y.reshape(1, BR, 8 * EMBED)

    @pl.when(pl.program_id(0) == 0)
    def _():
        sa_ref[...] = jnp.zeros_like(sa_ref)
        sb_ref[...] = jnp.zeros_like(sb_ref)

    rid = lax.broadcasted_iota(jnp.int32, (BR, 1), 0)
    real = (pl.program_id(0) * BR + rid) < NREAL
    ym = jnp.where(real, y, 0.0)
    sa_ref[...] += jnp.sum(ym, axis=0, keepdims=True)
    sb_ref[...] += jnp.sum(ym * ym, axis=0, keepdims=True)


def _bn2_body(y_ref, s1_ref, s2_ref, g_ref, be_ref, out_ref):
    mu = _fold8(s1_ref[...], EMBED) / N
    var = _fold8(s2_ref[...], EMBED) / N - mu * mu
    inv = lax.rsqrt(var + 1e-5)
    mu8 = jnp.concatenate([mu] * 8, axis=1)
    inv8 = jnp.concatenate([inv] * 8, axis=1)
    out = g_ref[...] * (y_ref[0] - mu8) * inv8 + be_ref[...]
    out_ref[...] = out.reshape(1, BR, 8 * EMBED)


def _full(shape):
    return pl.BlockSpec(shape, lambda i: tuple(0 for _ in shape))


def _pk3(lanes):
    return pl.BlockSpec((1, BR, lanes), lambda i: (i, 0, 0))


def _pk4(lead):
    return pl.BlockSpec((lead, 1, BR, 128), lambda i: (0, i, 0, 0))


def kernel(x, edge_index, node_type_ids, mask_faculty, mask_course,
           mask_section, mask_room, mask_timeslot, W_faculty, b_faculty,
           W_course, b_course, W_section, b_section, W_room, b_room,
           W_timeslot, b_timeslot, sage1_Wl, sage1_Wr, sage1_b,
           sage2_Wl, sage2_Wr, sage2_b, bn1_gamma, bn1_beta,
           bn2_gamma, bn2_beta):
    f32 = jnp.float32
    eye8 = np.eye(8, dtype=np.float32)
    Ws = [W_faculty, W_course, W_section, W_room, W_timeslot]
    bs = [b_faculty, b_course, b_section, b_room, b_timeslot]
    wpk = jnp.stack([jnp.kron(eye8, jnp.pad(w, ((0, 9 - w.shape[0]),
                                                (0, 0))))
                     for w in Ws])
    bpk = jnp.tile(jnp.stack(bs), (1, 8))
    i64 = np.eye(HIDDEN, dtype=np.float32)
    epk = jnp.asarray(np.stack(
        [np.kron(eye8, i64[:, p * 16:(p + 1) * 16]) for p in range(4)]))
    tmat_np = np.zeros((72, 8), np.float32)
    tmat_np[np.arange(8) * 9 + 8, np.arange(8)] = 1.0
    tmat = jnp.asarray(tmat_np)
    kmask = jnp.asarray(np.kron(eye8, np.ones((1, HIDDEN), np.float32)))
    q2 = np.arange(256)
    c2col = (q2 % 128 // 16) * 32 + (q2 // 128) * 16 + q2 % 16
    p256_np = np.zeros((256, 256), np.float32)
    p256_np[q2, c2col] = 1.0
    p256 = jnp.asarray(p256_np)
    q = jnp.arange(512)
    rowperm = (q % 128 // 16) * 64 + (q // 128) * 16 + q % 16
    wl1 = jnp.kron(eye8, sage1_Wl)[rowperm]
    wr1 = jnp.kron(eye8, sage1_Wr)
    b1 = jnp.tile(sage1_b, 8).reshape(1, 512)
    w2a = jnp.kron(eye8, sage2_Wl[:, :16])
    w2b = jnp.kron(eye8, sage2_Wl[:, 16:])
    wr2 = jnp.kron(eye8, sage2_Wr)
    b2 = jnp.tile(sage2_b, 8).reshape(1, 256)
    g512 = jnp.tile(bn1_gamma, 8).reshape(1, 512)
    be512 = jnp.tile(bn1_beta, 8).reshape(1, 512)
    g256 = jnp.tile(bn2_gamma, 8).reshape(1, 256)
    be256 = jnp.tile(bn2_beta, 8).reshape(1, 256)

    xt = jnp.concatenate(
        [x, node_type_ids.astype(f32).reshape(N, 1)], axis=1)
    xtp = jnp.pad(xt, ((0, NP - N), (0, 0))).reshape(GRID, BR, 72)

    npad = EP - E
    pad_src = jnp.asarray((np.arange(npad, dtype=np.int32) * 17) % N)
    pad_dst = jnp.asarray(
        N + (np.arange(npad, dtype=np.int32) % (NP - N)))
    src2d = jnp.concatenate([edge_index[0], pad_src]).reshape(RP_ALLOC, LN)
    dst2d = jnp.concatenate([edge_index[1], pad_dst]).reshape(RP_ALLOC, LN)
    z2d = jnp.zeros((NP, 16), f32)

    hc0, hc1, hc2, hc3 = pl.pallas_call(
        _proj_body,
        grid=(GRID,),
        in_specs=[_pk3(72), _full((5, 72, 512)), _full((5, 512)),
                  _full((72, 8)), _full((8, 512)),
                  _full((4, 512, 128))],
        out_specs=[_pk3(128)] * 4,
        out_shape=[jax.ShapeDtypeStruct((GRID, BR, 128), f32)] * 4,
    )(xtp, wpk, bpk, tmat, kmask, epk)

    agg1, degp = _make_segsum(4, True)(
        src2d, dst2d, z2d,
        hc0.reshape(NP, 16), hc1.reshape(NP, 16),
        hc2.reshape(NP, 16), hc3.reshape(NP, 16))
    agg1 = agg1.reshape(8, GRID, BR, 128)
    degp = degp.reshape(NC, GRID, BR, 128)

    y1, s1, s2 = pl.pallas_call(
        _combine1_body,
        grid=(GRID,),
        in_specs=[_pk3(72), _full((5, 72, 512)), _full((5, 512)),
                  _full((72, 8)), _full((8, 512)),
                  _pk4(8), _pk4(NC), _full((512, 512)),
                  _full((512, 512)), _full((1, 512))],
        out_specs=[_pk3(512), _full((1, 512)), _full((1, 512))],
        out_shape=[jax.ShapeDtypeStruct((GRID, BR, 512), f32),
                   jax.ShapeDtypeStruct((1, 512), f32),
                   jax.ShapeDtypeStruct((1, 512), f32)],
    )(xtp, wpk, bpk, tmat, kmask, agg1, degp, wl1, wr1, b1)

    g0, g1 = pl.pallas_call(
        _bn1_body,
        grid=(GRID,),
        in_specs=[_pk3(512), _full((1, 512)), _full((1, 512)),
                  _full((1, 512)), _full((1, 512)),
                  _full((512, 128)), _full((512, 128))],
        out_specs=[_pk3(128), _pk3(128)],
        out_shape=[jax.ShapeDtypeStruct((GRID, BR, 128), f32),
                   jax.ShapeDtypeStruct((GRID, BR, 128), f32)],
    )(y1, s1, s2, g512, be512, w2a, w2b)

    (agg2,) = _make_segsum(2, False)(src2d, dst2d, z2d,
                                     g0.reshape(NP, 16),
                                     g1.reshape(NP, 16))
    agg2 = agg2.reshape(4, GRID, BR, 128)

    y2, t1, t2 = pl.pallas_call(
        _combine2_body,
        grid=(GRID,),
        in_specs=[_pk3(512), _full((1, 512)), _full((1, 512)),
                  _full((1, 512)), _full((1, 512)),
                  _pk4(4), _pk4(NC), _full((512, 256)),
                  _full((1, 256)), _full((256, 256))],
        out_specs=[_pk3(256), _full((1, 256)), _full((1, 256))],
        out_shape=[jax.ShapeDtypeStruct((GRID, BR, 256), f32),
                   jax.ShapeDtypeStruct((1, 256), f32),
                   jax.ShapeDtypeStruct((1, 256), f32)],
    )(y1, s1, s2, g512, be512, agg2, degp, wr2, b2, p256)

    outp = pl.pallas_call(
        _bn2_body,
        grid=(GRID,),
        in_specs=[_pk3(256), _full((1, 256)), _full((1, 256)),
                  _full((1, 256)), _full((1, 256))],
        out_specs=_pk3(256),
        out_shape=jax.ShapeDtypeStruct((GRID, BR, 256), f32),
    )(y2, t1, t2, g256, be256)

    return outp.reshape(NP // 8, 8 * EMBED)[:NREAL].reshape(N, EMBED)
